# baseline (speedup 1.0000x reference)
import functools

import jax
import jax.numpy as jnp
from jax import lax
from jax.experimental import pallas as pl
from jax.experimental.pallas import tpu as pltpu
from jax.experimental.pallas import tpu_sc as plsc

N = 10000
E = 160000
D_IN = 256
D_H = 512
D_OUT = 256

NC = 2
NS = 16
L = 16

NPAD = 10240
COLS_PER_TILE = NPAD // NS

BATCH = 128
NB = -(-(E + N) // (NS * BATCH))
E2 = NS * NB * BATCH
ACC_ROWS = NPAD
ROWS_PER_TILE = N // NS
RED_BATCH = 25
IDXCHUNK = 12
NCHUNK = NB // IDXCHUNK
EPAD = E2 - (E + N)


def _rsqrt_newton(d):
  xi = plsc.bitcast(d, jnp.int32)
  i = jnp.int32(0x5F3759DF) - lax.shift_right_logical(xi, 1)
  r = plsc.bitcast(i, jnp.float32)
  for _ in range(3):
    r = r * (1.5 - 0.5 * d * r * r)
  return r


EPT_A = E // NS


def _stats_body(src_hbm, dst_hbm, dis_hbm, w_hbm, gidx_hbm, dst2_hbm,
                part_v, idx_v, idx2_v, disfull_v, red_v, chunk_v,
                ibuf_v, lbuf_v, lbuf2_v, pbuf_v,
                stage_sh, dis_sh):
  c = lax.axis_index("c")
  s = lax.axis_index("s")

  @pl.when(c == 0)
  def _():
    zeros16 = jnp.zeros((L,), jnp.float32)
    ones16 = jnp.ones((L,), jnp.float32)

    def zero_part(i, _):
      part_v[pl.ds(i * L, L)] = zeros16
      return 0
    lax.fori_loop(0, NPAD // L, zero_part, 0)

    pltpu.sync_copy(dst_hbm.at[pl.ds(s * EPT_A, EPT_A)], idx_v)

    def deg_step(i, _):
      d_idx = idx_v[pl.ds(i * L, L)]
      plsc.addupdate_scatter(part_v, [d_idx], ones16)
      return 0
    lax.fori_loop(0, EPT_A // L, deg_step, 0)

    pltpu.sync_copy(part_v, stage_sh.at[s])
    plsc.subcore_barrier()

    col0 = s * COLS_PER_TILE
    pltpu.sync_copy(stage_sh.at[:, pl.ds(col0, COLS_PER_TILE)], red_v)

    def dis_step(k, _):
      acc = red_v[0, pl.ds(k * L, L)]
      for t in range(1, NS):
        acc = acc + red_v[t, pl.ds(k * L, L)]
      chunk_v[pl.ds(k * L, L)] = _rsqrt_newton(acc + 1.0)
      return 0
    lax.fori_loop(0, COLS_PER_TILE // L, dis_step, 0)

    pltpu.sync_copy(chunk_v, dis_hbm.at[pl.ds(col0, COLS_PER_TILE)])
    pltpu.sync_copy(chunk_v, dis_sh.at[pl.ds(col0, COLS_PER_TILE)])
    plsc.subcore_barrier()

    pltpu.sync_copy(dis_sh, disfull_v)
    pltpu.sync_copy(src_hbm.at[pl.ds(s * EPT_A, EPT_A)], idx2_v)

    lax.fori_loop(0, NPAD // L, zero_part, 0)

    def t_step(i, _):
      d_idx = idx_v[pl.ds(i * L, L)]
      s_idx = idx2_v[pl.ds(i * L, L)]
      dvals = plsc.load_gather(disfull_v, [d_idx])
      plsc.addupdate_scatter(part_v, [s_idx], dvals)
      return 0
    lax.fori_loop(0, EPT_A // L, t_step, 0)

    pltpu.sync_copy(part_v, stage_sh.at[s])
    plsc.subcore_barrier()

    pltpu.sync_copy(stage_sh.at[:, pl.ds(col0, COLS_PER_TILE)], red_v)

    def w_step(k, _):
      acc = red_v[0, pl.ds(k * L, L)]
      for t in range(1, NS):
        acc = acc + red_v[t, pl.ds(k * L, L)]
      dv = disfull_v[pl.ds(col0 + k * L, L)]
      chunk_v[pl.ds(k * L, L)] = dv * (acc + dv)
      return 0
    lax.fori_loop(0, COLS_PER_TILE // L, w_step, 0)

    pltpu.sync_copy(chunk_v, w_hbm.at[pl.ds(col0, COLS_PER_TILE)])

    iota16 = lax.iota(jnp.int32, L)

    pltpu.sync_copy(idx_v, dst2_hbm.at[pl.ds(s * EPT_A, EPT_A)])
    for p in range(NC):
      def gi_step(i, _):
        ibuf_v[pl.ds(i * L, L)] = idx2_v[pl.ds(i * L, L)] * NC + p
        return 0
      lax.fori_loop(0, EPT_A // L, gi_step, 0)
      pltpu.sync_copy(ibuf_v, gidx_hbm.at[p, pl.ds(s * EPT_A, EPT_A)])

    def li_step(i, _):
      base = s * COLS_PER_TILE + i * L
      lbuf_v[pl.ds(i * L, L)] = base + iota16
      return 0
    lax.fori_loop(0, COLS_PER_TILE // L, li_step, 0)

    LLAST = N - (NS - 1) * COLS_PER_TILE

    @pl.when(s < NS - 1)
    def _():
      pltpu.sync_copy(lbuf_v, dst2_hbm.at[pl.ds(E + s * COLS_PER_TILE,
                                                COLS_PER_TILE)])

    @pl.when(s == NS - 1)
    def _():
      pltpu.sync_copy(lbuf_v.at[pl.ds(0, LLAST)],
                      dst2_hbm.at[pl.ds(E + s * COLS_PER_TILE, LLAST)])

    for p in range(NC):
      def gl_step(i, _):
        lbuf2_v[pl.ds(i * L, L)] = lbuf_v[pl.ds(i * L, L)] * NC + p
        return 0
      lax.fori_loop(0, COLS_PER_TILE // L, gl_step, 0)

      @pl.when(s < NS - 1)
      def _():
        pltpu.sync_copy(lbuf2_v, gidx_hbm.at[p, pl.ds(E + s * COLS_PER_TILE,
                                                      COLS_PER_TILE)])

      @pl.when(s == NS - 1)
      def _():
        pltpu.sync_copy(lbuf2_v.at[pl.ds(0, LLAST)],
                        gidx_hbm.at[p, pl.ds(E + s * COLS_PER_TILE, LLAST)])

    @pl.when(s == 0)
    def _():
      npad16 = jnp.full((L,), N, jnp.int32)

      def pz_step(i, _):
        pbuf_v[pl.ds(i * L, L)] = npad16
        return 0
      lax.fori_loop(0, EPAD // L, pz_step, 0)
      pltpu.sync_copy(pbuf_v, dst2_hbm.at[pl.ds(E + N, EPAD)])

      for p in range(NC):
        p16 = jnp.full((L,), p, jnp.int32)

        def pp_step(i, _):
          pbuf_v[pl.ds(i * L, L)] = p16
          return 0
        lax.fori_loop(0, EPAD // L, pp_step, 0)
        pltpu.sync_copy(pbuf_v, gidx_hbm.at[p, pl.ds(E + N, EPAD)])


@functools.partial(jax.jit, static_argnames=())
def _sc_stats(src, dst):
  mesh = plsc.VectorSubcoreMesh(core_axis_name="c", subcore_axis_name="s")
  f = pl.kernel(
      _stats_body,
      out_type=(
          jax.ShapeDtypeStruct((NPAD,), jnp.float32),
          jax.ShapeDtypeStruct((NPAD,), jnp.float32),
          jax.ShapeDtypeStruct((NC, E2), jnp.int32),
          jax.ShapeDtypeStruct((E2,), jnp.int32),
      ),
      mesh=mesh,
      scratch_types=[
          pltpu.VMEM((NPAD,), jnp.float32),
          pltpu.VMEM((EPT_A,), jnp.int32),
          pltpu.VMEM((EPT_A,), jnp.int32),
          pltpu.VMEM((NPAD,), jnp.float32),
          pltpu.VMEM((NS, COLS_PER_TILE), jnp.float32),
          pltpu.VMEM((COLS_PER_TILE,), jnp.float32),
          pltpu.VMEM((EPT_A,), jnp.int32),
          pltpu.VMEM((COLS_PER_TILE,), jnp.int32),
          pltpu.VMEM((COLS_PER_TILE,), jnp.int32),
          pltpu.VMEM((EPAD,), jnp.int32),
          pltpu.VMEM_SHARED((NS, NPAD), jnp.float32),
          pltpu.VMEM_SHARED((NPAD,), jnp.float32),
      ],
      compiler_params=pltpu.CompilerParams(use_tc_tiling_on_sc=False, needs_layout_passes=False),
  )
  return f(src, dst)


BM = 400


def _mm_body(x_ref, w_ref, dis_ref, out_ref):
  out_ref[...] = (dis_ref[...] * jnp.dot(
      x_ref[...].astype(jnp.bfloat16), w_ref[...].astype(jnp.bfloat16),
      preferred_element_type=jnp.float32)).astype(jnp.bfloat16)


def _tc_scaled_mm(x, W1, dis2):
  return pl.pallas_call(
      _mm_body,
      out_shape=jax.ShapeDtypeStruct((N, D_H), jnp.bfloat16),
      grid=(N // BM,),
      in_specs=[
          pl.BlockSpec((BM, D_IN), lambda i: (i, 0)),
          pl.BlockSpec((D_IN, D_H), lambda i: (0, 0)),
          pl.BlockSpec((BM, 1), lambda i: (i, 0)),
      ],
      out_specs=pl.BlockSpec((BM, D_H), lambda i: (i, 0)),
  )(x, W1, dis2)


SLAB = D_H // NC


def _msg_body(y2_hbm, gidx_hbm, didx_hbm, out_hbm,
              gidx_v, didx_v, buf0, buf1, zbuf,
              sem0, sem1, sem2, sem3, acc_sh):
  c = lax.axis_index("c")
  s = lax.axis_index("s")
  zeros32 = jnp.zeros((2 * L,), jnp.bfloat16)

  def zb(i, _):
    for f in range(SLAB // (2 * L)):
      zbuf[i, pl.ds(f * 2 * L, 2 * L)] = zeros32
    return 0
  lax.fori_loop(0, 16, zb, 0)

  for z in range(COLS_PER_TILE // 16):
    pltpu.sync_copy(zbuf, acc_sh.at[pl.ds(s * COLS_PER_TILE + z * 16, 16)])
  plsc.subcore_barrier()

  def fire_g(j, buf, sem):
    pltpu.async_copy(y2_hbm.at[gidx_v.at[j]], buf, sem)

  def wait_g_fire_s(j, buf, gsem, ssem):
    pltpu.make_async_copy(y2_hbm.at[gidx_v.at[j]], buf, gsem).wait()
    pltpu.async_copy(buf, acc_sh.at[didx_v.at[j]], ssem, add=True)

  def wait_s(j, buf, ssem):
    pltpu.make_async_copy(buf, acc_sh.at[didx_v.at[j]], ssem).wait()

  def chunk_step(ic, _):
    j0 = s * NB + ic * IDXCHUNK
    pltpu.sync_copy(gidx_hbm.at[c, pl.ds(j0, IDXCHUNK)], gidx_v)
    pltpu.sync_copy(didx_hbm.at[pl.ds(j0, IDXCHUNK)], didx_v)
    fire_g(0, buf0, sem0)
    fire_g(1, buf1, sem1)

    def edge_step(t, _):
      j = t * 2
      wait_g_fire_s(j, buf0, sem0, sem2)
      wait_g_fire_s(j + 1, buf1, sem1, sem3)
      wait_s(j, buf0, sem2)

      @pl.when(j + 2 < IDXCHUNK)
      def _():
        fire_g(j + 2, buf0, sem0)

      wait_s(j + 1, buf1, sem3)

      @pl.when(j + 3 < IDXCHUNK)
      def _():
        fire_g(j + 3, buf1, sem1)

      return 0
    lax.fori_loop(0, IDXCHUNK // 2, edge_step, 0)
    return 0
  lax.fori_loop(0, NCHUNK, chunk_step, 0)

  plsc.subcore_barrier()

  pltpu.sync_copy(acc_sh.at[pl.ds(s * COLS_PER_TILE, COLS_PER_TILE)],
                  out_hbm.at[c, pl.ds(s * COLS_PER_TILE, COLS_PER_TILE)])


def _sc_message(y2, gidx2, didx):
  mesh = plsc.VectorSubcoreMesh(core_axis_name="c", subcore_axis_name="s")
  f = pl.kernel(
      _msg_body,
      out_type=jax.ShapeDtypeStruct((NC, NPAD, SLAB), jnp.bfloat16),
      mesh=mesh,
      scratch_types=[
          pltpu.VMEM((IDXCHUNK, BATCH), jnp.int32),
          pltpu.VMEM((IDXCHUNK, BATCH), jnp.int32),
          pltpu.VMEM((BATCH, SLAB), jnp.bfloat16),
          pltpu.VMEM((BATCH, SLAB), jnp.bfloat16),
          pltpu.VMEM((16, SLAB), jnp.bfloat16),
          pltpu.SemaphoreType.DMA,
          pltpu.SemaphoreType.DMA,
          pltpu.SemaphoreType.DMA,
          pltpu.SemaphoreType.DMA,
          pltpu.VMEM_SHARED((ACC_ROWS, SLAB), jnp.bfloat16),
      ],
      compiler_params=pltpu.CompilerParams(use_tc_tiling_on_sc=False, needs_layout_passes=False),
  )
  return f(y2, gidx2, didx)


BMD = 400


def _red_body(acc_ref, dis_ref, w_ref, b1_ref, w2_ref, b2_ref, out_ref, s_scr):
  i = pl.program_id(0)

  @pl.when(i == 0)
  def _():
    s_scr[...] = jnp.zeros_like(s_scr)

  dis = dis_ref[...]
  wv = w_ref[...]
  for p in range(NC):
    h = jnp.maximum(
        dis * acc_ref[p].astype(jnp.float32) + b1_ref[p], 0.0)
    s_scr[p] += jnp.sum(wv * h, axis=0, keepdims=True)

  @pl.when(i == (N // BMD) - 1)
  def _():
    acc = jnp.zeros((1, D_OUT), jnp.float32)
    for p in range(NC):
      acc = acc + jnp.dot(s_scr[p], w2_ref[p],
                          preferred_element_type=jnp.float32)
    out_ref[...] = acc * (1.0 / N) + b2_ref[...]


def _tc_reduce(acc2, dis2, w2col, b1r, W2r, b2r):
  return pl.pallas_call(
      _red_body,
      out_shape=jax.ShapeDtypeStruct((1, D_OUT), jnp.float32),
      grid=(N // BMD,),
      in_specs=[
          pl.BlockSpec((NC, BMD, SLAB), lambda i: (0, i, 0)),
          pl.BlockSpec((BMD, 1), lambda i: (i, 0)),
          pl.BlockSpec((BMD, 1), lambda i: (i, 0)),
          pl.BlockSpec((NC, 1, SLAB), lambda i: (0, 0, 0)),
          pl.BlockSpec((NC, SLAB, D_OUT), lambda i: (0, 0, 0)),
          pl.BlockSpec((1, D_OUT), lambda i: (0, 0)),
      ],
      out_specs=pl.BlockSpec((1, D_OUT), lambda i: (0, 0)),
      scratch_shapes=[pltpu.VMEM((NC, 1, SLAB), jnp.float32)],
  )(acc2, dis2, w2col, b1r, W2r, b2r)


def kernel(x, edge_index, W1, b1, W2, b2):
  src = edge_index[0]
  dst = edge_index[1]

  dis_pad, w_pad, gidx2, dst2 = _sc_stats(src, dst)
  dis = dis_pad[:N]

  y = _tc_scaled_mm(x, W1, dis.reshape(N, 1))
  y2 = y.reshape(N * NC, SLAB)

  acc2 = _sc_message(y2,
                     gidx2.reshape(NC, NS * NB, BATCH),
                     dst2.reshape(NS * NB, BATCH))

  out = _tc_reduce(acc2,
                   dis.reshape(N, 1),
                   w_pad[:N].reshape(N, 1),
                   b1.reshape(NC, 1, SLAB),
                   W2.reshape(NC, SLAB, D_OUT),
                   b2.reshape(1, D_OUT))
  return out.reshape(D_OUT)


# speedup vs baseline: 1.0539x; 1.0539x over previous
import functools

import jax
import jax.numpy as jnp
from jax import lax
from jax.experimental import pallas as pl
from jax.experimental.pallas import tpu as pltpu
from jax.experimental.pallas import tpu_sc as plsc

N = 10000
E = 160000
D_IN = 256
D_H = 512
D_OUT = 256

NC = 2
NS = 16
L = 16

NPAD = 10240
COLS_PER_TILE = NPAD // NS

BATCH = 128
NB = -(-(E + N) // (NS * BATCH))
E2 = NS * NB * BATCH
ACC_ROWS = NPAD
ROWS_PER_TILE = N // NS
RED_BATCH = 25
IDXCHUNK = 12
NCHUNK = NB // IDXCHUNK
EPAD = E2 - (E + N)


def _rsqrt_newton(d):
  xi = plsc.bitcast(d, jnp.int32)
  i = jnp.int32(0x5F3759DF) - lax.shift_right_logical(xi, 1)
  r = plsc.bitcast(i, jnp.float32)
  for _ in range(3):
    r = r * (1.5 - 0.5 * d * r * r)
  return r


EPT_A = E // NS


def _stats_body(src_hbm, dst_hbm, dis_hbm, w_hbm, gidx_hbm, dst2_hbm,
                part_v, idx_v, idx2_v, disfull_v, red_v, chunk_v,
                lbuf_v, pbuf_v,
                stage_sh, dis_sh):
  c = lax.axis_index("c")
  s = lax.axis_index("s")

  @pl.when(c == 0)
  def _():
    zeros16 = jnp.zeros((L,), jnp.float32)
    ones16 = jnp.ones((L,), jnp.float32)

    def zero_part(i, _):
      part_v[pl.ds(i * L, L)] = zeros16
      return 0
    lax.fori_loop(0, NPAD // L, zero_part, 0)

    pltpu.sync_copy(dst_hbm.at[pl.ds(s * EPT_A, EPT_A)], idx_v)

    def deg_step(i, _):
      d_idx = idx_v[pl.ds(i * L, L)]
      plsc.addupdate_scatter(part_v, [d_idx], ones16)
      return 0
    lax.fori_loop(0, EPT_A // L, deg_step, 0)

    pltpu.sync_copy(part_v, stage_sh.at[s])
    plsc.subcore_barrier()

    col0 = s * COLS_PER_TILE
    pltpu.sync_copy(stage_sh.at[:, pl.ds(col0, COLS_PER_TILE)], red_v)

    def dis_step(k, _):
      acc = red_v[0, pl.ds(k * L, L)]
      for t in range(1, NS):
        acc = acc + red_v[t, pl.ds(k * L, L)]
      chunk_v[pl.ds(k * L, L)] = _rsqrt_newton(acc + 1.0)
      return 0
    lax.fori_loop(0, COLS_PER_TILE // L, dis_step, 0)

    pltpu.sync_copy(chunk_v, dis_hbm.at[pl.ds(col0, COLS_PER_TILE)])
    pltpu.sync_copy(chunk_v, dis_sh.at[pl.ds(col0, COLS_PER_TILE)])
    plsc.subcore_barrier()

    pltpu.sync_copy(dis_sh, disfull_v)
    pltpu.sync_copy(src_hbm.at[pl.ds(s * EPT_A, EPT_A)], idx2_v)

    lax.fori_loop(0, NPAD // L, zero_part, 0)

    def t_step(i, _):
      d_idx = idx_v[pl.ds(i * L, L)]
      s_idx = idx2_v[pl.ds(i * L, L)]
      dvals = plsc.load_gather(disfull_v, [d_idx])
      plsc.addupdate_scatter(part_v, [s_idx], dvals)
      return 0
    lax.fori_loop(0, EPT_A // L, t_step, 0)

    pltpu.sync_copy(part_v, stage_sh.at[s])
    plsc.subcore_barrier()

    pltpu.sync_copy(stage_sh.at[:, pl.ds(col0, COLS_PER_TILE)], red_v)

    def w_step(k, _):
      acc = red_v[0, pl.ds(k * L, L)]
      for t in range(1, NS):
        acc = acc + red_v[t, pl.ds(k * L, L)]
      dv = disfull_v[pl.ds(col0 + k * L, L)]
      chunk_v[pl.ds(k * L, L)] = dv * (acc + dv)
      return 0
    lax.fori_loop(0, COLS_PER_TILE // L, w_step, 0)

    pltpu.sync_copy(chunk_v, w_hbm.at[pl.ds(col0, COLS_PER_TILE)])

    iota16 = lax.iota(jnp.int32, L)

    pltpu.sync_copy(idx_v, dst2_hbm.at[pl.ds(s * EPT_A, EPT_A)])
    pltpu.sync_copy(idx2_v, gidx_hbm.at[pl.ds(s * EPT_A, EPT_A)])

    def li_step(i, _):
      base = s * COLS_PER_TILE + i * L
      lbuf_v[pl.ds(i * L, L)] = base + iota16
      return 0
    lax.fori_loop(0, COLS_PER_TILE // L, li_step, 0)

    LLAST = N - (NS - 1) * COLS_PER_TILE

    @pl.when(s < NS - 1)
    def _():
      pltpu.sync_copy(lbuf_v, dst2_hbm.at[pl.ds(E + s * COLS_PER_TILE,
                                                COLS_PER_TILE)])
      pltpu.sync_copy(lbuf_v, gidx_hbm.at[pl.ds(E + s * COLS_PER_TILE,
                                                COLS_PER_TILE)])

    @pl.when(s == NS - 1)
    def _():
      pltpu.sync_copy(lbuf_v.at[pl.ds(0, LLAST)],
                      dst2_hbm.at[pl.ds(E + s * COLS_PER_TILE, LLAST)])
      pltpu.sync_copy(lbuf_v.at[pl.ds(0, LLAST)],
                      gidx_hbm.at[pl.ds(E + s * COLS_PER_TILE, LLAST)])

    @pl.when(s == 0)
    def _():
      npad16 = jnp.full((L,), N, jnp.int32)

      def pz_step(i, _):
        pbuf_v[pl.ds(i * L, L)] = npad16
        return 0
      lax.fori_loop(0, EPAD // L, pz_step, 0)
      pltpu.sync_copy(pbuf_v, dst2_hbm.at[pl.ds(E + N, EPAD)])

      zero16 = jnp.zeros((L,), jnp.int32)

      def pp_step(i, _):
        pbuf_v[pl.ds(i * L, L)] = zero16
        return 0
      lax.fori_loop(0, EPAD // L, pp_step, 0)
      pltpu.sync_copy(pbuf_v, gidx_hbm.at[pl.ds(E + N, EPAD)])


@functools.partial(jax.jit, static_argnames=())
def _sc_stats(src, dst):
  mesh = plsc.VectorSubcoreMesh(core_axis_name="c", subcore_axis_name="s")
  f = pl.kernel(
      _stats_body,
      out_type=(
          jax.ShapeDtypeStruct((NPAD,), jnp.float32),
          jax.ShapeDtypeStruct((NPAD,), jnp.float32),
          jax.ShapeDtypeStruct((E2,), jnp.int32),
          jax.ShapeDtypeStruct((E2,), jnp.int32),
      ),
      mesh=mesh,
      scratch_types=[
          pltpu.VMEM((NPAD,), jnp.float32),
          pltpu.VMEM((EPT_A,), jnp.int32),
          pltpu.VMEM((EPT_A,), jnp.int32),
          pltpu.VMEM((NPAD,), jnp.float32),
          pltpu.VMEM((NS, COLS_PER_TILE), jnp.float32),
          pltpu.VMEM((COLS_PER_TILE,), jnp.float32),
          pltpu.VMEM((COLS_PER_TILE,), jnp.int32),
          pltpu.VMEM((EPAD,), jnp.int32),
          pltpu.VMEM_SHARED((NS, NPAD), jnp.float32),
          pltpu.VMEM_SHARED((NPAD,), jnp.float32),
      ],
      compiler_params=pltpu.CompilerParams(use_tc_tiling_on_sc=False, needs_layout_passes=False),
  )
  return f(src, dst)


BM = 400


def _mm_body(x_ref, w_ref, dis_ref, outa_ref, outb_ref):
  y = dis_ref[...] * jnp.dot(
      x_ref[...].astype(jnp.bfloat16), w_ref[...].astype(jnp.bfloat16),
      preferred_element_type=jnp.float32)
  outa_ref[...] = y[:, :D_H // 2].astype(jnp.bfloat16)
  outb_ref[...] = y[:, D_H // 2:].astype(jnp.bfloat16)


def _tc_scaled_mm(x, W1, dis2):
  half = jax.ShapeDtypeStruct((N, D_H // 2), jnp.bfloat16)
  return pl.pallas_call(
      _mm_body,
      out_shape=(half, half),
      grid=(N // BM,),
      in_specs=[
          pl.BlockSpec((BM, D_IN), lambda i: (i, 0)),
          pl.BlockSpec((D_IN, D_H), lambda i: (0, 0)),
          pl.BlockSpec((BM, 1), lambda i: (i, 0)),
      ],
      out_specs=(pl.BlockSpec((BM, D_H // 2), lambda i: (i, 0)),
                 pl.BlockSpec((BM, D_H // 2), lambda i: (i, 0))),
  )(x, W1, dis2)


SLAB = D_H // NC


def _msg_body(ya_hbm, yb_hbm, gidx_hbm, didx_hbm, out_hbm,
              gidx_v, didx_v, buf0, buf1, zbuf,
              sem0, sem1, sem2, sem3, acc_sh):
  c = lax.axis_index("c")
  s = lax.axis_index("s")
  zeros32 = jnp.zeros((2 * L,), jnp.bfloat16)

  def zb(i, _):
    for f in range(SLAB // (2 * L)):
      zbuf[i, pl.ds(f * 2 * L, 2 * L)] = zeros32
    return 0
  lax.fori_loop(0, 16, zb, 0)

  for z in range(COLS_PER_TILE // 16):
    pltpu.sync_copy(zbuf, acc_sh.at[pl.ds(s * COLS_PER_TILE + z * 16, 16)])
  plsc.subcore_barrier()

  def run_edges(y_hbm):
    def fire_g(j, buf, sem):
      pltpu.async_copy(y_hbm.at[gidx_v.at[pl.ds(j * BATCH, BATCH)]], buf, sem)

    def wait_g_fire_s(j, buf, gsem, ssem):
      pltpu.make_async_copy(
          y_hbm.at[gidx_v.at[pl.ds(j * BATCH, BATCH)]], buf, gsem).wait()
      pltpu.async_copy(
          buf, acc_sh.at[didx_v.at[pl.ds(j * BATCH, BATCH)]], ssem, add=True)

    def wait_s(j, buf, ssem):
      pltpu.make_async_copy(
          buf, acc_sh.at[didx_v.at[pl.ds(j * BATCH, BATCH)]], ssem).wait()

    def chunk_step(ic, _):
      e0 = (s * NB + ic * IDXCHUNK) * BATCH
      pltpu.sync_copy(gidx_hbm.at[pl.ds(e0, IDXCHUNK * BATCH)], gidx_v)
      pltpu.sync_copy(didx_hbm.at[pl.ds(e0, IDXCHUNK * BATCH)], didx_v)
      fire_g(0, buf0, sem0)
      fire_g(1, buf1, sem1)

      def edge_step(t, _):
        j = t * 2
        wait_g_fire_s(j, buf0, sem0, sem2)
        wait_g_fire_s(j + 1, buf1, sem1, sem3)
        wait_s(j, buf0, sem2)

        @pl.when(j + 2 < IDXCHUNK)
        def _():
          fire_g(j + 2, buf0, sem0)

        wait_s(j + 1, buf1, sem3)

        @pl.when(j + 3 < IDXCHUNK)
        def _():
          fire_g(j + 3, buf1, sem1)

        return 0
      lax.fori_loop(0, IDXCHUNK // 2, edge_step, 0)
      return 0
    lax.fori_loop(0, NCHUNK, chunk_step, 0)

  @pl.when(c == 0)
  def _():
    run_edges(ya_hbm)

  @pl.when(c == 1)
  def _():
    run_edges(yb_hbm)

  plsc.subcore_barrier()

  pltpu.sync_copy(acc_sh.at[pl.ds(s * COLS_PER_TILE, COLS_PER_TILE)],
                  out_hbm.at[c, pl.ds(s * COLS_PER_TILE, COLS_PER_TILE)])


def _sc_message(ya, yb, gidx, didx):
  mesh = plsc.VectorSubcoreMesh(core_axis_name="c", subcore_axis_name="s")
  f = pl.kernel(
      _msg_body,
      out_type=jax.ShapeDtypeStruct((NC, NPAD, SLAB), jnp.bfloat16),
      mesh=mesh,
      scratch_types=[
          pltpu.VMEM((IDXCHUNK * BATCH,), jnp.int32),
          pltpu.VMEM((IDXCHUNK * BATCH,), jnp.int32),
          pltpu.VMEM((BATCH, SLAB), jnp.bfloat16),
          pltpu.VMEM((BATCH, SLAB), jnp.bfloat16),
          pltpu.VMEM((16, SLAB), jnp.bfloat16),
          pltpu.SemaphoreType.DMA,
          pltpu.SemaphoreType.DMA,
          pltpu.SemaphoreType.DMA,
          pltpu.SemaphoreType.DMA,
          pltpu.VMEM_SHARED((ACC_ROWS, SLAB), jnp.bfloat16),
      ],
      compiler_params=pltpu.CompilerParams(use_tc_tiling_on_sc=False, needs_layout_passes=False),
  )
  return f(ya, yb, gidx, didx)


BMD = 400


def _red_body(acc_ref, dis_ref, w_ref, b1_ref, w2_ref, b2_ref, out_ref, s_scr):
  i = pl.program_id(0)

  @pl.when(i == 0)
  def _():
    s_scr[...] = jnp.zeros_like(s_scr)

  dis = dis_ref[...]
  wv = w_ref[...]
  for p in range(NC):
    h = jnp.maximum(
        dis * acc_ref[p].astype(jnp.float32) + b1_ref[p], 0.0)
    s_scr[p] += jnp.sum(wv * h, axis=0, keepdims=True)

  @pl.when(i == (N // BMD) - 1)
  def _():
    acc = jnp.zeros((1, D_OUT), jnp.float32)
    for p in range(NC):
      acc = acc + jnp.dot(s_scr[p], w2_ref[p],
                          preferred_element_type=jnp.float32)
    out_ref[...] = acc * (1.0 / N) + b2_ref[...]


def _tc_reduce(acc2, dis2, w2col, b1r, W2r, b2r):
  return pl.pallas_call(
      _red_body,
      out_shape=jax.ShapeDtypeStruct((1, D_OUT), jnp.float32),
      grid=(N // BMD,),
      in_specs=[
          pl.BlockSpec((NC, BMD, SLAB), lambda i: (0, i, 0)),
          pl.BlockSpec((BMD, 1), lambda i: (i, 0)),
          pl.BlockSpec((BMD, 1), lambda i: (i, 0)),
          pl.BlockSpec((NC, 1, SLAB), lambda i: (0, 0, 0)),
          pl.BlockSpec((NC, SLAB, D_OUT), lambda i: (0, 0, 0)),
          pl.BlockSpec((1, D_OUT), lambda i: (0, 0)),
      ],
      out_specs=pl.BlockSpec((1, D_OUT), lambda i: (0, 0)),
      scratch_shapes=[pltpu.VMEM((NC, 1, SLAB), jnp.float32)],
  )(acc2, dis2, w2col, b1r, W2r, b2r)


def kernel(x, edge_index, W1, b1, W2, b2):
  src = edge_index[0]
  dst = edge_index[1]

  dis_pad, w_pad, gidx, dst2 = _sc_stats(src, dst)
  dis = dis_pad[:N]

  ya, yb = _tc_scaled_mm(x, W1, dis.reshape(N, 1))

  acc2 = _sc_message(ya, yb, gidx, dst2)

  out = _tc_reduce(acc2,
                   dis.reshape(N, 1),
                   w_pad[:N].reshape(N, 1),
                   b1.reshape(NC, 1, SLAB),
                   W2.reshape(NC, SLAB, D_OUT),
                   b2.reshape(1, D_OUT))
  return out.reshape(D_OUT)


# speedup vs baseline: 1.1049x; 1.0484x over previous
import functools

import jax
import jax.numpy as jnp
from jax import lax
from jax.experimental import pallas as pl
from jax.experimental.pallas import tpu as pltpu
from jax.experimental.pallas import tpu_sc as plsc

N = 10000
E = 160000
D_IN = 256
D_H = 512
D_OUT = 256

NC = 2
NS = 16
L = 16

NPAD = 10240
COLS_PER_TILE = NPAD // NS

BATCH = 128
NB = -(-(E + N) // (NS * BATCH))
E2 = NS * NB * BATCH
ACC_ROWS = NPAD
ROWS_PER_TILE = N // NS
RED_BATCH = 25
IDXCHUNK = 12
NCHUNK = NB // IDXCHUNK
EPAD = E2 - (E + N)


def _rsqrt_newton(d):
  xi = plsc.bitcast(d, jnp.int32)
  i = jnp.int32(0x5F3759DF) - lax.shift_right_logical(xi, 1)
  r = plsc.bitcast(i, jnp.float32)
  for _ in range(3):
    r = r * (1.5 - 0.5 * d * r * r)
  return r


EPT_A = E // NS


def _stats_body(src_hbm, dst_hbm, dis_hbm, w_hbm, ga_hbm, gb_hbm, dst2_hbm,
                part_v, idx_v, idx2_v, disfull_v, red_v, chunk_v,
                ibuf_v, lbuf_v, lbuf2_v, pbuf_v,
                stage_sh, dis_sh):
  gidx_hbms = (ga_hbm, gb_hbm)
  c = lax.axis_index("c")
  s = lax.axis_index("s")

  @pl.when(c == 0)
  def _():
    zeros16 = jnp.zeros((L,), jnp.float32)
    ones16 = jnp.ones((L,), jnp.float32)

    def zero_part(i, _):
      part_v[pl.ds(i * L, L)] = zeros16
      return 0
    lax.fori_loop(0, NPAD // L, zero_part, 0)

    pltpu.sync_copy(dst_hbm.at[pl.ds(s * EPT_A, EPT_A)], idx_v)

    def deg_step(i, _):
      d_idx = idx_v[pl.ds(i * L, L)]
      plsc.addupdate_scatter(part_v, [d_idx], ones16)
      return 0
    lax.fori_loop(0, EPT_A // L, deg_step, 0)

    pltpu.sync_copy(part_v, stage_sh.at[s])
    plsc.subcore_barrier()

    col0 = s * COLS_PER_TILE
    pltpu.sync_copy(stage_sh.at[:, pl.ds(col0, COLS_PER_TILE)], red_v)

    def dis_step(k, _):
      acc = red_v[0, pl.ds(k * L, L)]
      for t in range(1, NS):
        acc = acc + red_v[t, pl.ds(k * L, L)]
      chunk_v[pl.ds(k * L, L)] = _rsqrt_newton(acc + 1.0)
      return 0
    lax.fori_loop(0, COLS_PER_TILE // L, dis_step, 0)

    pltpu.sync_copy(chunk_v, dis_hbm.at[pl.ds(col0, COLS_PER_TILE)])
    pltpu.sync_copy(chunk_v, dis_sh.at[pl.ds(col0, COLS_PER_TILE)])
    plsc.subcore_barrier()

    pltpu.sync_copy(dis_sh, disfull_v)
    pltpu.sync_copy(src_hbm.at[pl.ds(s * EPT_A, EPT_A)], idx2_v)

    lax.fori_loop(0, NPAD // L, zero_part, 0)

    def t_step(i, _):
      d_idx = idx_v[pl.ds(i * L, L)]
      s_idx = idx2_v[pl.ds(i * L, L)]
      dvals = plsc.load_gather(disfull_v, [d_idx])
      plsc.addupdate_scatter(part_v, [s_idx], dvals)
      return 0
    lax.fori_loop(0, EPT_A // L, t_step, 0)

    pltpu.sync_copy(part_v, stage_sh.at[s])
    plsc.subcore_barrier()

    pltpu.sync_copy(stage_sh.at[:, pl.ds(col0, COLS_PER_TILE)], red_v)

    def w_step(k, _):
      acc = red_v[0, pl.ds(k * L, L)]
      for t in range(1, NS):
        acc = acc + red_v[t, pl.ds(k * L, L)]
      dv = disfull_v[pl.ds(col0 + k * L, L)]
      chunk_v[pl.ds(k * L, L)] = dv * (acc + dv)
      return 0
    lax.fori_loop(0, COLS_PER_TILE // L, w_step, 0)

    pltpu.sync_copy(chunk_v, w_hbm.at[pl.ds(col0, COLS_PER_TILE)])

    iota16 = lax.iota(jnp.int32, L)

    pltpu.sync_copy(idx_v, dst2_hbm.at[pl.ds(s * EPT_A, EPT_A)])
    for p in range(NC):
      def gi_step(i, _):
        ibuf_v[pl.ds(i * L, L)] = idx2_v[pl.ds(i * L, L)] * NC + p
        return 0
      lax.fori_loop(0, EPT_A // L, gi_step, 0)
      pltpu.sync_copy(ibuf_v, gidx_hbms[p].at[pl.ds(s * EPT_A, EPT_A)])

    def li_step(i, _):
      base = s * COLS_PER_TILE + i * L
      lbuf_v[pl.ds(i * L, L)] = base + iota16
      return 0
    lax.fori_loop(0, COLS_PER_TILE // L, li_step, 0)

    LLAST = N - (NS - 1) * COLS_PER_TILE

    @pl.when(s < NS - 1)
    def _():
      pltpu.sync_copy(lbuf_v, dst2_hbm.at[pl.ds(E + s * COLS_PER_TILE,
                                                COLS_PER_TILE)])

    @pl.when(s == NS - 1)
    def _():
      pltpu.sync_copy(lbuf_v.at[pl.ds(0, LLAST)],
                      dst2_hbm.at[pl.ds(E + s * COLS_PER_TILE, LLAST)])

    for p in range(NC):
      def gl_step(i, _):
        lbuf2_v[pl.ds(i * L, L)] = lbuf_v[pl.ds(i * L, L)] * NC + p
        return 0
      lax.fori_loop(0, COLS_PER_TILE // L, gl_step, 0)

      @pl.when(s < NS - 1)
      def _():
        pltpu.sync_copy(lbuf2_v, gidx_hbms[p].at[pl.ds(E + s * COLS_PER_TILE,
                                                       COLS_PER_TILE)])

      @pl.when(s == NS - 1)
      def _():
        pltpu.sync_copy(lbuf2_v.at[pl.ds(0, LLAST)],
                        gidx_hbms[p].at[pl.ds(E + s * COLS_PER_TILE, LLAST)])

    @pl.when(s == 0)
    def _():
      npad16 = jnp.full((L,), N, jnp.int32)

      def pz_step(i, _):
        pbuf_v[pl.ds(i * L, L)] = npad16
        return 0
      lax.fori_loop(0, EPAD // L, pz_step, 0)
      pltpu.sync_copy(pbuf_v, dst2_hbm.at[pl.ds(E + N, EPAD)])

      for p in range(NC):
        p16 = jnp.full((L,), p, jnp.int32)

        def pp_step(i, _):
          pbuf_v[pl.ds(i * L, L)] = p16
          return 0
        lax.fori_loop(0, EPAD // L, pp_step, 0)
        pltpu.sync_copy(pbuf_v, gidx_hbms[p].at[pl.ds(E + N, EPAD)])


@functools.partial(jax.jit, static_argnames=())
def _sc_stats(src, dst):
  mesh = plsc.VectorSubcoreMesh(core_axis_name="c", subcore_axis_name="s")
  f = pl.kernel(
      _stats_body,
      out_type=(
          jax.ShapeDtypeStruct((NPAD,), jnp.float32),
          jax.ShapeDtypeStruct((NPAD,), jnp.float32),
          jax.ShapeDtypeStruct((E2,), jnp.int32),
          jax.ShapeDtypeStruct((E2,), jnp.int32),
          jax.ShapeDtypeStruct((E2,), jnp.int32),
      ),
      mesh=mesh,
      scratch_types=[
          pltpu.VMEM((NPAD,), jnp.float32),
          pltpu.VMEM((EPT_A,), jnp.int32),
          pltpu.VMEM((EPT_A,), jnp.int32),
          pltpu.VMEM((NPAD,), jnp.float32),
          pltpu.VMEM((NS, COLS_PER_TILE), jnp.float32),
          pltpu.VMEM((COLS_PER_TILE,), jnp.float32),
          pltpu.VMEM((EPT_A,), jnp.int32),
          pltpu.VMEM((COLS_PER_TILE,), jnp.int32),
          pltpu.VMEM((COLS_PER_TILE,), jnp.int32),
          pltpu.VMEM((EPAD,), jnp.int32),
          pltpu.VMEM_SHARED((NS, NPAD), jnp.float32),
          pltpu.VMEM_SHARED((NPAD,), jnp.float32),
      ],
      compiler_params=pltpu.CompilerParams(use_tc_tiling_on_sc=False, needs_layout_passes=False),
  )
  return f(src, dst)


BM = 400


def _mm_body(x_ref, w_ref, dis_ref, out_ref):
  y = dis_ref[...] * jnp.dot(
      x_ref[...].astype(jnp.bfloat16), w_ref[...].astype(jnp.bfloat16),
      preferred_element_type=jnp.float32)
  out_ref[...] = y.astype(jnp.bfloat16).reshape(2 * BM, D_H // 2)


def _tc_scaled_mm(x, W1, dis2):
  return pl.pallas_call(
      _mm_body,
      out_shape=jax.ShapeDtypeStruct((NC * N, D_H // 2), jnp.bfloat16),
      grid=(N // BM,),
      in_specs=[
          pl.BlockSpec((BM, D_IN), lambda i: (i, 0)),
          pl.BlockSpec((D_IN, D_H), lambda i: (0, 0)),
          pl.BlockSpec((BM, 1), lambda i: (i, 0)),
      ],
      out_specs=pl.BlockSpec((2 * BM, D_H // 2), lambda i: (i, 0)),
  )(x, W1, dis2)


SLAB = D_H // NC


def _msg_body(y2_hbm, ga_hbm, gb_hbm, didx_hbm, out_hbm,
              gidx_v, didx_v, buf0, buf1, zbuf,
              sem0, sem1, sem2, sem3, acc_sh):
  c = lax.axis_index("c")
  s = lax.axis_index("s")
  zeros32 = jnp.zeros((2 * L,), jnp.bfloat16)

  def zb(i, _):
    for f in range(SLAB // (2 * L)):
      zbuf[i, pl.ds(f * 2 * L, 2 * L)] = zeros32
    return 0
  lax.fori_loop(0, 16, zb, 0)

  for z in range(COLS_PER_TILE // 16):
    pltpu.sync_copy(zbuf, acc_sh.at[pl.ds(s * COLS_PER_TILE + z * 16, 16)])
  plsc.subcore_barrier()

  def fire_g(j, buf, sem):
    pltpu.async_copy(y2_hbm.at[gidx_v.at[pl.ds(j * BATCH, BATCH)]], buf, sem)

  def wait_g_fire_s(j, buf, gsem, ssem):
    pltpu.make_async_copy(
        y2_hbm.at[gidx_v.at[pl.ds(j * BATCH, BATCH)]], buf, gsem).wait()
    pltpu.async_copy(
        buf, acc_sh.at[didx_v.at[pl.ds(j * BATCH, BATCH)]], ssem, add=True)

  def wait_s(j, buf, ssem):
    pltpu.make_async_copy(
        buf, acc_sh.at[didx_v.at[pl.ds(j * BATCH, BATCH)]], ssem).wait()

  def chunk_step(ic, _):
    e0 = (s * NB + ic * IDXCHUNK) * BATCH

    @pl.when(c == 0)
    def _():
      pltpu.sync_copy(ga_hbm.at[pl.ds(e0, IDXCHUNK * BATCH)], gidx_v)

    @pl.when(c == 1)
    def _():
      pltpu.sync_copy(gb_hbm.at[pl.ds(e0, IDXCHUNK * BATCH)], gidx_v)

    pltpu.sync_copy(didx_hbm.at[pl.ds(e0, IDXCHUNK * BATCH)], didx_v)
    fire_g(0, buf0, sem0)
    fire_g(1, buf1, sem1)

    def edge_step(t, _):
      j = t * 2
      wait_g_fire_s(j, buf0, sem0, sem2)
      wait_g_fire_s(j + 1, buf1, sem1, sem3)
      wait_s(j, buf0, sem2)

      @pl.when(j + 2 < IDXCHUNK)
      def _():
        fire_g(j + 2, buf0, sem0)

      wait_s(j + 1, buf1, sem3)

      @pl.when(j + 3 < IDXCHUNK)
      def _():
        fire_g(j + 3, buf1, sem1)

      return 0
    lax.fori_loop(0, IDXCHUNK // 2, edge_step, 0)
    return 0
  lax.fori_loop(0, NCHUNK, chunk_step, 0)

  plsc.subcore_barrier()

  pltpu.sync_copy(acc_sh.at[pl.ds(s * COLS_PER_TILE, COLS_PER_TILE)],
                  out_hbm.at[c, pl.ds(s * COLS_PER_TILE, COLS_PER_TILE)])


def _sc_message(y2, ga, gb, didx):
  mesh = plsc.VectorSubcoreMesh(core_axis_name="c", subcore_axis_name="s")
  f = pl.kernel(
      _msg_body,
      out_type=jax.ShapeDtypeStruct((NC, NPAD, SLAB), jnp.bfloat16),
      mesh=mesh,
      scratch_types=[
          pltpu.VMEM((IDXCHUNK * BATCH,), jnp.int32),
          pltpu.VMEM((IDXCHUNK * BATCH,), jnp.int32),
          pltpu.VMEM((BATCH, SLAB), jnp.bfloat16),
          pltpu.VMEM((BATCH, SLAB), jnp.bfloat16),
          pltpu.VMEM((16, SLAB), jnp.bfloat16),
          pltpu.SemaphoreType.DMA,
          pltpu.SemaphoreType.DMA,
          pltpu.SemaphoreType.DMA,
          pltpu.SemaphoreType.DMA,
          pltpu.VMEM_SHARED((ACC_ROWS, SLAB), jnp.bfloat16),
      ],
      compiler_params=pltpu.CompilerParams(use_tc_tiling_on_sc=False, needs_layout_passes=False),
  )
  return f(y2, ga, gb, didx)


BMD = 400


HALF = SLAB // 2


def _red_body(acc_ref, dis_ref, w_ref, b1_ref, w2_ref, b2_ref, out_ref, s_scr):
  i = pl.program_id(0)

  @pl.when(i == 0)
  def _():
    s_scr[...] = jnp.zeros_like(s_scr)

  dis = dis_ref[...]
  wv = w_ref[...]
  for p in range(NC):
    h = jnp.maximum(
        dis * acc_ref[p].astype(jnp.float32) + b1_ref[p], 0.0)
    s_scr[p] += jnp.sum(wv * h, axis=0, keepdims=True)

  @pl.when(i == (N // BMD) - 1)
  def _():
    acc = jnp.zeros((1, D_OUT), jnp.float32)
    for p in range(NC):
      acc = acc + jnp.dot(s_scr[p], w2_ref[p],
                          preferred_element_type=jnp.float32)
    out_ref[...] = acc * (1.0 / N) + b2_ref[...]


def _tc_reduce(acc2, dis2, w2col, b1r, W2r, b2r):
  return pl.pallas_call(
      _red_body,
      out_shape=jax.ShapeDtypeStruct((1, D_OUT), jnp.float32),
      grid=(N // BMD,),
      in_specs=[
          pl.BlockSpec((NC, BMD, SLAB), lambda i: (0, i, 0)),
          pl.BlockSpec((BMD, 1), lambda i: (i, 0)),
          pl.BlockSpec((BMD, 1), lambda i: (i, 0)),
          pl.BlockSpec((NC, 1, SLAB), lambda i: (0, 0, 0)),
          pl.BlockSpec((NC, SLAB, D_OUT), lambda i: (0, 0, 0)),
          pl.BlockSpec((1, D_OUT), lambda i: (0, 0)),
      ],
      out_specs=pl.BlockSpec((1, D_OUT), lambda i: (0, 0)),
      scratch_shapes=[pltpu.VMEM((NC, 1, SLAB), jnp.float32)],
  )(acc2, dis2, w2col, b1r, W2r, b2r)


def kernel(x, edge_index, W1, b1, W2, b2):
  src = edge_index[0]
  dst = edge_index[1]

  dis_pad, w_pad, ga, gb, dst2 = _sc_stats(src, dst)
  dis = dis_pad[:N]

  y2 = _tc_scaled_mm(x, W1, dis.reshape(N, 1))

  acc2 = _sc_message(y2, ga, gb, dst2)

  out = _tc_reduce(acc2,
                   dis.reshape(N, 1),
                   w_pad[:N].reshape(N, 1),
                   b1.reshape(NC, 1, SLAB),
                   W2.reshape(NC, SLAB, D_OUT),
                   b2.reshape(1, D_OUT))
  return out.reshape(D_OUT)


# speedup vs baseline: 1.1120x; 1.0063x over previous
import functools

import jax
import jax.numpy as jnp
from jax import lax
from jax.experimental import pallas as pl
from jax.experimental.pallas import tpu as pltpu
from jax.experimental.pallas import tpu_sc as plsc

N = 10000
E = 160000
D_IN = 256
D_H = 512
D_OUT = 256

NC = 2
NS = 16
L = 16

NPAD = 10240
COLS_PER_TILE = NPAD // NS

BATCH = 128
NB = -(-(E + N) // (NS * BATCH))
E2 = NS * NB * BATCH
ACC_ROWS = NPAD
ROWS_PER_TILE = N // NS
RED_BATCH = 25
IDXCHUNK = 12
NCHUNK = NB // IDXCHUNK
EPAD = E2 - (E + N)


def _rsqrt_newton(d):
  xi = plsc.bitcast(d, jnp.int32)
  i = jnp.int32(0x5F3759DF) - lax.shift_right_logical(xi, 1)
  r = plsc.bitcast(i, jnp.float32)
  for _ in range(3):
    r = r * (1.5 - 0.5 * d * r * r)
  return r


EPT_A = E // NS


def _stats_body(src_hbm, dst_hbm, dis_hbm, w_hbm, g2_hbm, didx2_hbm,
                part_v, idx_v, idx2_v, disfull_v, red_v, chunk_v,
                e3s_v, e3d_v, ga2_v, gb2_v, dd2_v,
                stage_sh, dis_sh):
  c = lax.axis_index("c")
  s = lax.axis_index("s")

  @pl.when(c == 0)
  def _():
    zeros16 = jnp.zeros((L,), jnp.float32)
    ones16 = jnp.ones((L,), jnp.float32)

    def zero_part(i, _):
      part_v[pl.ds(i * L, L)] = zeros16
      return 0
    lax.fori_loop(0, NPAD // L, zero_part, 0)

    pltpu.sync_copy(dst_hbm.at[pl.ds(s * EPT_A, EPT_A)], idx_v)

    def deg_step(i, _):
      d_idx = idx_v[pl.ds(i * L, L)]
      plsc.addupdate_scatter(part_v, [d_idx], ones16)
      return 0
    lax.fori_loop(0, EPT_A // L, deg_step, 0)

    pltpu.sync_copy(part_v, stage_sh.at[s])
    plsc.subcore_barrier()

    col0 = s * COLS_PER_TILE
    pltpu.sync_copy(stage_sh.at[:, pl.ds(col0, COLS_PER_TILE)], red_v)

    def dis_step(k, _):
      acc = red_v[0, pl.ds(k * L, L)]
      for t in range(1, NS):
        acc = acc + red_v[t, pl.ds(k * L, L)]
      chunk_v[pl.ds(k * L, L)] = _rsqrt_newton(acc + 1.0)
      return 0
    lax.fori_loop(0, COLS_PER_TILE // L, dis_step, 0)

    pltpu.sync_copy(chunk_v, dis_hbm.at[pl.ds(col0, COLS_PER_TILE)])
    pltpu.sync_copy(chunk_v, dis_sh.at[pl.ds(col0, COLS_PER_TILE)])
    plsc.subcore_barrier()

    pltpu.sync_copy(dis_sh, disfull_v)
    pltpu.sync_copy(src_hbm.at[pl.ds(s * EPT_A, EPT_A)], idx2_v)

    lax.fori_loop(0, NPAD // L, zero_part, 0)

    def t_step(i, _):
      d_idx = idx_v[pl.ds(i * L, L)]
      s_idx = idx2_v[pl.ds(i * L, L)]
      dvals = plsc.load_gather(disfull_v, [d_idx])
      plsc.addupdate_scatter(part_v, [s_idx], dvals)
      return 0
    lax.fori_loop(0, EPT_A // L, t_step, 0)

    pltpu.sync_copy(part_v, stage_sh.at[s])
    plsc.subcore_barrier()

    pltpu.sync_copy(stage_sh.at[:, pl.ds(col0, COLS_PER_TILE)], red_v)

    def w_step(k, _):
      acc = red_v[0, pl.ds(k * L, L)]
      for t in range(1, NS):
        acc = acc + red_v[t, pl.ds(k * L, L)]
      dv = disfull_v[pl.ds(col0 + k * L, L)]
      chunk_v[pl.ds(k * L, L)] = dv * (acc + dv)
      return 0
    lax.fori_loop(0, COLS_PER_TILE // L, w_step, 0)

    pltpu.sync_copy(chunk_v, w_hbm.at[pl.ds(col0, COLS_PER_TILE)])

    iota16 = lax.iota(jnp.int32, L)
    EPT3 = NB * BATCH
    g0 = s * EPT3

    @pl.when(s * EPT3 + EPT3 <= E)
    def _():
      pltpu.sync_copy(src_hbm.at[pl.ds(g0, EPT3)], e3s_v)
      pltpu.sync_copy(dst_hbm.at[pl.ds(g0, EPT3)], e3d_v)

    @pl.when((s * EPT3 < E) & (s * EPT3 + EPT3 > E))
    def _():
      rem = E - 14 * EPT3
      pltpu.sync_copy(src_hbm.at[pl.ds(14 * EPT3, rem)],
                      e3s_v.at[pl.ds(0, rem)])
      pltpu.sync_copy(dst_hbm.at[pl.ds(14 * EPT3, rem)],
                      e3d_v.at[pl.ds(0, rem)])

    def asm_step(i, _):
      g = g0 + i * L + iota16
      sv = e3s_v[pl.ds(i * L, L)]
      dv = e3d_v[pl.ds(i * L, L)]
      selfid = g - E
      src_val = jnp.where(g < E, sv, jnp.where(g < E + N, selfid, 0))
      dst_val = jnp.where(g < E, dv, jnp.where(g < E + N, selfid, N))
      j = i // (BATCH // L)
      k = (i % (BATCH // L)) * L
      ga2_v[j, pl.ds(k, L)] = src_val * NC
      gb2_v[j, pl.ds(k, L)] = src_val * NC + 1
      dd2_v[j, pl.ds(k, L)] = dst_val
      return 0
    lax.fori_loop(0, EPT3 // L, asm_step, 0)

    pltpu.sync_copy(ga2_v, g2_hbm.at[0, pl.ds(s * NB, NB)])
    pltpu.sync_copy(gb2_v, g2_hbm.at[1, pl.ds(s * NB, NB)])
    pltpu.sync_copy(dd2_v, didx2_hbm.at[pl.ds(s * NB, NB)])


@functools.partial(jax.jit, static_argnames=())
def _sc_stats(src, dst):
  mesh = plsc.VectorSubcoreMesh(core_axis_name="c", subcore_axis_name="s")
  f = pl.kernel(
      _stats_body,
      out_type=(
          jax.ShapeDtypeStruct((NPAD,), jnp.float32),
          jax.ShapeDtypeStruct((NPAD,), jnp.float32),
          jax.ShapeDtypeStruct((NC, NS * NB, BATCH), jnp.int32),
          jax.ShapeDtypeStruct((NS * NB, BATCH), jnp.int32),
      ),
      mesh=mesh,
      scratch_types=[
          pltpu.VMEM((NPAD,), jnp.float32),
          pltpu.VMEM((EPT_A,), jnp.int32),
          pltpu.VMEM((EPT_A,), jnp.int32),
          pltpu.VMEM((NPAD,), jnp.float32),
          pltpu.VMEM((NS, COLS_PER_TILE), jnp.float32),
          pltpu.VMEM((COLS_PER_TILE,), jnp.float32),
          pltpu.VMEM((NB * BATCH,), jnp.int32),
          pltpu.VMEM((NB * BATCH,), jnp.int32),
          pltpu.VMEM((NB, BATCH), jnp.int32),
          pltpu.VMEM((NB, BATCH), jnp.int32),
          pltpu.VMEM((NB, BATCH), jnp.int32),
          pltpu.VMEM_SHARED((NS, NPAD), jnp.float32),
          pltpu.VMEM_SHARED((NPAD,), jnp.float32),
      ],
      compiler_params=pltpu.CompilerParams(use_tc_tiling_on_sc=False, needs_layout_passes=False),
  )
  return f(src, dst)


BM = 400


def _mm_body(x_ref, w_ref, dis_ref, out_ref):
  y = dis_ref[...] * jnp.dot(
      x_ref[...].astype(jnp.bfloat16), w_ref[...].astype(jnp.bfloat16),
      preferred_element_type=jnp.float32)
  out_ref[...] = y.astype(jnp.bfloat16).reshape(2 * BM, D_H // 2)


def _tc_scaled_mm(x, W1, dis2):
  return pl.pallas_call(
      _mm_body,
      out_shape=jax.ShapeDtypeStruct((NC * N, D_H // 2), jnp.bfloat16),
      grid=(N // BM,),
      in_specs=[
          pl.BlockSpec((BM, D_IN), lambda i: (i, 0)),
          pl.BlockSpec((D_IN, D_H), lambda i: (0, 0)),
          pl.BlockSpec((BM, 1), lambda i: (i, 0)),
      ],
      out_specs=pl.BlockSpec((2 * BM, D_H // 2), lambda i: (i, 0)),
  )(x, W1, dis2)


SLAB = D_H // NC


def _msg_body(y2_hbm, g2_hbm, didx2_hbm, out_hbm,
              gidx_v, didx_v, buf0, buf1, zbuf,
              sem0, sem1, sem2, sem3, acc_sh):
  c = lax.axis_index("c")
  s = lax.axis_index("s")
  g_hbm = g2_hbm.at[c]
  zeros32 = jnp.zeros((2 * L,), jnp.bfloat16)

  def zb(i, _):
    for f in range(SLAB // (2 * L)):
      zbuf[i, pl.ds(f * 2 * L, 2 * L)] = zeros32
    return 0
  lax.fori_loop(0, 16, zb, 0)

  for z in range(COLS_PER_TILE // 16):
    pltpu.sync_copy(zbuf, acc_sh.at[pl.ds(s * COLS_PER_TILE + z * 16, 16)])
  plsc.subcore_barrier()

  def fire_g(j, buf, sem):
    pltpu.async_copy(y2_hbm.at[gidx_v.at[j]], buf, sem)

  def wait_g_fire_s(j, buf, gsem, ssem):
    pltpu.make_async_copy(y2_hbm.at[gidx_v.at[j]], buf, gsem).wait()
    pltpu.async_copy(buf, acc_sh.at[didx_v.at[j]], ssem, add=True)

  def wait_s(j, buf, ssem):
    pltpu.make_async_copy(buf, acc_sh.at[didx_v.at[j]], ssem).wait()

  def chunk_step(ic, _):
    j0 = s * NB + ic * IDXCHUNK
    pltpu.sync_copy(g_hbm.at[pl.ds(j0, IDXCHUNK)], gidx_v)
    pltpu.sync_copy(didx2_hbm.at[pl.ds(j0, IDXCHUNK)], didx_v)
    fire_g(0, buf0, sem0)
    fire_g(1, buf1, sem1)

    def edge_step(t, _):
      j = t * 2
      wait_g_fire_s(j, buf0, sem0, sem2)
      wait_g_fire_s(j + 1, buf1, sem1, sem3)
      wait_s(j, buf0, sem2)

      @pl.when(j + 2 < IDXCHUNK)
      def _():
        fire_g(j + 2, buf0, sem0)

      wait_s(j + 1, buf1, sem3)

      @pl.when(j + 3 < IDXCHUNK)
      def _():
        fire_g(j + 3, buf1, sem1)

      return 0
    lax.fori_loop(0, IDXCHUNK // 2, edge_step, 0)
    return 0
  lax.fori_loop(0, NCHUNK, chunk_step, 0)

  plsc.subcore_barrier()

  pltpu.sync_copy(acc_sh.at[pl.ds(s * COLS_PER_TILE, COLS_PER_TILE)],
                  out_hbm.at[c, pl.ds(s * COLS_PER_TILE, COLS_PER_TILE)])


def _sc_message(y2, g2, didx2):
  mesh = plsc.VectorSubcoreMesh(core_axis_name="c", subcore_axis_name="s")
  f = pl.kernel(
      _msg_body,
      out_type=jax.ShapeDtypeStruct((NC, NPAD, SLAB), jnp.bfloat16),
      mesh=mesh,
      scratch_types=[
          pltpu.VMEM((IDXCHUNK, BATCH), jnp.int32),
          pltpu.VMEM((IDXCHUNK, BATCH), jnp.int32),
          pltpu.VMEM((BATCH, SLAB), jnp.bfloat16),
          pltpu.VMEM((BATCH, SLAB), jnp.bfloat16),
          pltpu.VMEM((16, SLAB), jnp.bfloat16),
          pltpu.SemaphoreType.DMA,
          pltpu.SemaphoreType.DMA,
          pltpu.SemaphoreType.DMA,
          pltpu.SemaphoreType.DMA,
          pltpu.VMEM_SHARED((ACC_ROWS, SLAB), jnp.bfloat16),
      ],
      compiler_params=pltpu.CompilerParams(use_tc_tiling_on_sc=False, needs_layout_passes=False),
  )
  return f(y2, g2, didx2)


BMD = 400


HALF = SLAB // 2


def _red_body(acc_ref, dis_ref, w_ref, b1_ref, w2_ref, b2_ref, out_ref, s_scr):
  i = pl.program_id(0)

  @pl.when(i == 0)
  def _():
    s_scr[...] = jnp.zeros_like(s_scr)

  dis = dis_ref[...]
  wv = w_ref[...]
  for p in range(NC):
    h = jnp.maximum(
        dis * acc_ref[p].astype(jnp.float32) + b1_ref[p], 0.0)
    s_scr[p] += jnp.sum(wv * h, axis=0, keepdims=True)

  @pl.when(i == (N // BMD) - 1)
  def _():
    acc = jnp.zeros((1, D_OUT), jnp.float32)
    for p in range(NC):
      acc = acc + jnp.dot(s_scr[p], w2_ref[p],
                          preferred_element_type=jnp.float32)
    out_ref[...] = acc * (1.0 / N) + b2_ref[...]


def _tc_reduce(acc2, dis2, w2col, b1r, W2r, b2r):
  return pl.pallas_call(
      _red_body,
      out_shape=jax.ShapeDtypeStruct((1, D_OUT), jnp.float32),
      grid=(N // BMD,),
      in_specs=[
          pl.BlockSpec((NC, BMD, SLAB), lambda i: (0, i, 0)),
          pl.BlockSpec((BMD, 1), lambda i: (i, 0)),
          pl.BlockSpec((BMD, 1), lambda i: (i, 0)),
          pl.BlockSpec((NC, 1, SLAB), lambda i: (0, 0, 0)),
          pl.BlockSpec((NC, SLAB, D_OUT), lambda i: (0, 0, 0)),
          pl.BlockSpec((1, D_OUT), lambda i: (0, 0)),
      ],
      out_specs=pl.BlockSpec((1, D_OUT), lambda i: (0, 0)),
      scratch_shapes=[pltpu.VMEM((NC, 1, SLAB), jnp.float32)],
  )(acc2, dis2, w2col, b1r, W2r, b2r)


def kernel(x, edge_index, W1, b1, W2, b2):
  src = edge_index[0]
  dst = edge_index[1]

  dis_pad, w_pad, g2, didx2 = _sc_stats(src, dst)
  dis = dis_pad[:N]

  y2 = _tc_scaled_mm(x, W1, dis.reshape(N, 1))

  acc2 = _sc_message(y2, g2, didx2)

  out = _tc_reduce(acc2,
                   dis.reshape(N, 1),
                   w_pad[:N].reshape(N, 1),
                   b1.reshape(NC, 1, SLAB),
                   W2.reshape(NC, SLAB, D_OUT),
                   b2.reshape(1, D_OUT))
  return out.reshape(D_OUT)


# speedup vs baseline: 1.1523x; 1.0363x over previous
import functools

import jax
import jax.numpy as jnp
from jax import lax
from jax.experimental import pallas as pl
from jax.experimental.pallas import tpu as pltpu
from jax.experimental.pallas import tpu_sc as plsc

N = 10000
E = 160000
D_IN = 256
D_H = 512
D_OUT = 256

NC = 2
NS = 16
L = 16

NPAD = 10240
COLS_PER_TILE = NPAD // NS

BATCH = 128
NB = -(-(E + N) // (NS * BATCH))
E2 = NS * NB * BATCH
ACC_ROWS = NPAD
ROWS_PER_TILE = N // NS
RED_BATCH = 25
IDXCHUNK = 12
NCHUNK = NB // IDXCHUNK
EPAD = E2 - (E + N)


def _rsqrt_newton(d):
  xi = plsc.bitcast(d, jnp.int32)
  i = jnp.int32(0x5F3759DF) - lax.shift_right_logical(xi, 1)
  r = plsc.bitcast(i, jnp.float32)
  for _ in range(3):
    r = r * (1.5 - 0.5 * d * r * r)
  return r


EPT_A = E // NS


def _stats_body(src_hbm, dst_hbm, dis_hbm, w_hbm,
                part_v, idx_v, idx2_v, disfull_v, red_v, chunk_v,
                stage_sh, dis_sh):
  c = lax.axis_index("c")
  s = lax.axis_index("s")

  @pl.when(c == 0)
  def _():
    zeros16 = jnp.zeros((L,), jnp.float32)
    ones16 = jnp.ones((L,), jnp.float32)

    def zero_part(i, _):
      part_v[pl.ds(i * L, L)] = zeros16
      return 0
    lax.fori_loop(0, NPAD // L, zero_part, 0)

    pltpu.sync_copy(dst_hbm.at[pl.ds(s * EPT_A, EPT_A)], idx_v)

    def deg_step(i, _):
      d_idx = idx_v[pl.ds(i * L, L)]
      plsc.addupdate_scatter(part_v, [d_idx], ones16)
      return 0
    lax.fori_loop(0, EPT_A // L, deg_step, 0)

    pltpu.sync_copy(part_v, stage_sh.at[s])
    plsc.subcore_barrier()

    col0 = s * COLS_PER_TILE
    pltpu.sync_copy(stage_sh.at[:, pl.ds(col0, COLS_PER_TILE)], red_v)

    def dis_step(k, _):
      acc = red_v[0, pl.ds(k * L, L)]
      for t in range(1, NS):
        acc = acc + red_v[t, pl.ds(k * L, L)]
      chunk_v[pl.ds(k * L, L)] = _rsqrt_newton(acc + 1.0)
      return 0
    lax.fori_loop(0, COLS_PER_TILE // L, dis_step, 0)

    pltpu.sync_copy(chunk_v, dis_hbm.at[pl.ds(col0, COLS_PER_TILE)])
    pltpu.sync_copy(chunk_v, dis_sh.at[pl.ds(col0, COLS_PER_TILE)])
    plsc.subcore_barrier()

    pltpu.sync_copy(dis_sh, disfull_v)
    pltpu.sync_copy(src_hbm.at[pl.ds(s * EPT_A, EPT_A)], idx2_v)

    lax.fori_loop(0, NPAD // L, zero_part, 0)

    def t_step(i, _):
      d_idx = idx_v[pl.ds(i * L, L)]
      s_idx = idx2_v[pl.ds(i * L, L)]
      dvals = plsc.load_gather(disfull_v, [d_idx])
      plsc.addupdate_scatter(part_v, [s_idx], dvals)
      return 0
    lax.fori_loop(0, EPT_A // L, t_step, 0)

    pltpu.sync_copy(part_v, stage_sh.at[s])
    plsc.subcore_barrier()

    pltpu.sync_copy(stage_sh.at[:, pl.ds(col0, COLS_PER_TILE)], red_v)

    def w_step(k, _):
      acc = red_v[0, pl.ds(k * L, L)]
      for t in range(1, NS):
        acc = acc + red_v[t, pl.ds(k * L, L)]
      dv = disfull_v[pl.ds(col0 + k * L, L)]
      chunk_v[pl.ds(k * L, L)] = dv * (acc + dv)
      return 0
    lax.fori_loop(0, COLS_PER_TILE // L, w_step, 0)

    pltpu.sync_copy(chunk_v, w_hbm.at[pl.ds(col0, COLS_PER_TILE)])


@functools.partial(jax.jit, static_argnames=())
def _sc_stats(src, dst):
  mesh = plsc.VectorSubcoreMesh(core_axis_name="c", subcore_axis_name="s")
  f = pl.kernel(
      _stats_body,
      out_type=(
          jax.ShapeDtypeStruct((NPAD,), jnp.float32),
          jax.ShapeDtypeStruct((NPAD,), jnp.float32),
      ),
      mesh=mesh,
      scratch_types=[
          pltpu.VMEM((NPAD,), jnp.float32),
          pltpu.VMEM((EPT_A,), jnp.int32),
          pltpu.VMEM((EPT_A,), jnp.int32),
          pltpu.VMEM((NPAD,), jnp.float32),
          pltpu.VMEM((NS, COLS_PER_TILE), jnp.float32),
          pltpu.VMEM((COLS_PER_TILE,), jnp.float32),
          pltpu.VMEM_SHARED((NS, NPAD), jnp.float32),
          pltpu.VMEM_SHARED((NPAD,), jnp.float32),
      ],
      compiler_params=pltpu.CompilerParams(use_tc_tiling_on_sc=False, needs_layout_passes=False),
  )
  return f(src, dst)


BM = 400


def _mm_body(x_ref, w_ref, dis_ref, out_ref):
  y = dis_ref[...] * jnp.dot(
      x_ref[...].astype(jnp.bfloat16), w_ref[...].astype(jnp.bfloat16),
      preferred_element_type=jnp.float32)
  out_ref[...] = y.astype(jnp.bfloat16).reshape(2 * BM, D_H // 2)


def _tc_scaled_mm(x, W1, dis2):
  return pl.pallas_call(
      _mm_body,
      out_shape=jax.ShapeDtypeStruct((NC * N, D_H // 2), jnp.bfloat16),
      grid=(N // BM,),
      in_specs=[
          pl.BlockSpec((BM, D_IN), lambda i: (i, 0)),
          pl.BlockSpec((D_IN, D_H), lambda i: (0, 0)),
          pl.BlockSpec((BM, 1), lambda i: (i, 0)),
      ],
      out_specs=pl.BlockSpec((2 * BM, D_H // 2), lambda i: (i, 0)),
  )(x, W1, dis2)


SLAB = D_H // NC


def _msg_body(src_hbm, dst_hbm, y2_hbm, out_hbm,
              gidx_v, didx_v, buf0, buf1, zbuf,
              sem0, sem1, sem2, sem3, acc_sh):
  c = lax.axis_index("c")
  s = lax.axis_index("s")
  iota16 = lax.iota(jnp.int32, L)
  CH = IDXCHUNK * BATCH
  zeros32 = jnp.zeros((2 * L,), jnp.bfloat16)

  def zb(i, _):
    for f in range(SLAB // (2 * L)):
      zbuf[i, pl.ds(f * 2 * L, 2 * L)] = zeros32
    return 0
  lax.fori_loop(0, 16, zb, 0)

  for z in range(COLS_PER_TILE // 16):
    pltpu.sync_copy(zbuf, acc_sh.at[pl.ds(s * COLS_PER_TILE + z * 16, 16)])
  plsc.subcore_barrier()

  def fire_g(j, buf, sem):
    pltpu.async_copy(y2_hbm.at[gidx_v.at[pl.ds(j * BATCH, BATCH)]], buf, sem)

  def wait_g_fire_s(j, buf, gsem, ssem):
    pltpu.make_async_copy(
        y2_hbm.at[gidx_v.at[pl.ds(j * BATCH, BATCH)]], buf, gsem).wait()
    pltpu.async_copy(
        buf, acc_sh.at[didx_v.at[pl.ds(j * BATCH, BATCH)]], ssem, add=True)

  def wait_s(j, buf, ssem):
    pltpu.make_async_copy(
        buf, acc_sh.at[didx_v.at[pl.ds(j * BATCH, BATCH)]], ssem).wait()

  def chunk_step(ic, _):
    w0 = (s * NB + ic * IDXCHUNK) * BATCH

    @pl.when(w0 + CH <= E)
    def _():
      pltpu.sync_copy(src_hbm.at[pl.ds(w0, CH)], gidx_v)
      pltpu.sync_copy(dst_hbm.at[pl.ds(w0, CH)], didx_v)

    @pl.when((w0 < E) & (w0 + CH > E))
    def _():
      rem = E % CH
      pltpu.sync_copy(src_hbm.at[pl.ds(E - rem, rem)],
                      gidx_v.at[pl.ds(0, rem)])
      pltpu.sync_copy(dst_hbm.at[pl.ds(E - rem, rem)],
                      didx_v.at[pl.ds(0, rem)])

    def asm_step(i, _):
      g = w0 + i * L + iota16
      sv = gidx_v[pl.ds(i * L, L)]
      dv = didx_v[pl.ds(i * L, L)]
      selfid = g - E
      src_val = jnp.where(g < E, sv, jnp.where(g < E + N, selfid, 0))
      dst_val = jnp.where(g < E, dv, jnp.where(g < E + N, selfid, N))
      gidx_v[pl.ds(i * L, L)] = src_val * NC + c
      didx_v[pl.ds(i * L, L)] = dst_val
      return 0
    lax.fori_loop(0, CH // L, asm_step, 0)
    fire_g(0, buf0, sem0)
    fire_g(1, buf1, sem1)

    def edge_step(t, _):
      j = t * 2
      wait_g_fire_s(j, buf0, sem0, sem2)
      wait_g_fire_s(j + 1, buf1, sem1, sem3)
      wait_s(j, buf0, sem2)

      @pl.when(j + 2 < IDXCHUNK)
      def _():
        fire_g(j + 2, buf0, sem0)

      wait_s(j + 1, buf1, sem3)

      @pl.when(j + 3 < IDXCHUNK)
      def _():
        fire_g(j + 3, buf1, sem1)

      return 0
    lax.fori_loop(0, IDXCHUNK // 2, edge_step, 0)
    return 0
  lax.fori_loop(0, NCHUNK, chunk_step, 0)

  plsc.subcore_barrier()

  pltpu.sync_copy(acc_sh.at[pl.ds(s * COLS_PER_TILE, COLS_PER_TILE)],
                  out_hbm.at[c, pl.ds(s * COLS_PER_TILE, COLS_PER_TILE)])


def _sc_message(src, dst, y2):
  mesh = plsc.VectorSubcoreMesh(core_axis_name="c", subcore_axis_name="s")
  f = pl.kernel(
      _msg_body,
      out_type=jax.ShapeDtypeStruct((NC, NPAD, SLAB), jnp.bfloat16),
      mesh=mesh,
      scratch_types=[
          pltpu.VMEM((IDXCHUNK * BATCH,), jnp.int32),
          pltpu.VMEM((IDXCHUNK * BATCH,), jnp.int32),
          pltpu.VMEM((BATCH, SLAB), jnp.bfloat16),
          pltpu.VMEM((BATCH, SLAB), jnp.bfloat16),
          pltpu.VMEM((16, SLAB), jnp.bfloat16),
          pltpu.SemaphoreType.DMA,
          pltpu.SemaphoreType.DMA,
          pltpu.SemaphoreType.DMA,
          pltpu.SemaphoreType.DMA,
          pltpu.VMEM_SHARED((ACC_ROWS, SLAB), jnp.bfloat16),
      ],
      compiler_params=pltpu.CompilerParams(use_tc_tiling_on_sc=False, needs_layout_passes=False),
  )
  return f(src, dst, y2)


BMD = 400


HALF = SLAB // 2


def _red_body(acc_ref, dis_ref, w_ref, b1_ref, w2_ref, b2_ref, out_ref, s_scr):
  i = pl.program_id(0)

  @pl.when(i == 0)
  def _():
    s_scr[...] = jnp.zeros_like(s_scr)

  dis = dis_ref[...]
  wv = w_ref[...]
  for p in range(NC):
    h = jnp.maximum(
        dis * acc_ref[p].astype(jnp.float32) + b1_ref[p], 0.0)
    s_scr[p] += jnp.sum(wv * h, axis=0, keepdims=True)

  @pl.when(i == (N // BMD) - 1)
  def _():
    acc = jnp.zeros((1, D_OUT), jnp.float32)
    for p in range(NC):
      acc = acc + jnp.dot(s_scr[p], w2_ref[p],
                          preferred_element_type=jnp.float32)
    out_ref[...] = acc * (1.0 / N) + b2_ref[...]


def _tc_reduce(acc2, dis2, w2col, b1r, W2r, b2r):
  return pl.pallas_call(
      _red_body,
      out_shape=jax.ShapeDtypeStruct((1, D_OUT), jnp.float32),
      grid=(N // BMD,),
      in_specs=[
          pl.BlockSpec((NC, BMD, SLAB), lambda i: (0, i, 0)),
          pl.BlockSpec((BMD, 1), lambda i: (i, 0)),
          pl.BlockSpec((BMD, 1), lambda i: (i, 0)),
          pl.BlockSpec((NC, 1, SLAB), lambda i: (0, 0, 0)),
          pl.BlockSpec((NC, SLAB, D_OUT), lambda i: (0, 0, 0)),
          pl.BlockSpec((1, D_OUT), lambda i: (0, 0)),
      ],
      out_specs=pl.BlockSpec((1, D_OUT), lambda i: (0, 0)),
      scratch_shapes=[pltpu.VMEM((NC, 1, SLAB), jnp.float32)],
  )(acc2, dis2, w2col, b1r, W2r, b2r)


def kernel(x, edge_index, W1, b1, W2, b2):
  src = edge_index[0]
  dst = edge_index[1]

  dis_pad, w_pad = _sc_stats(src, dst)
  dis = dis_pad[:N]

  y2 = _tc_scaled_mm(x, W1, dis.reshape(N, 1))

  acc2 = _sc_message(src, dst, y2)

  out = _tc_reduce(acc2,
                   dis.reshape(N, 1),
                   w_pad[:N].reshape(N, 1),
                   b1.reshape(NC, 1, SLAB),
                   W2.reshape(NC, SLAB, D_OUT),
                   b2.reshape(1, D_OUT))
  return out.reshape(D_OUT)


# speedup vs baseline: 1.1618x; 1.0082x over previous
import functools

import jax
import jax.numpy as jnp
from jax import lax
from jax.experimental import pallas as pl
from jax.experimental.pallas import tpu as pltpu
from jax.experimental.pallas import tpu_sc as plsc

N = 10000
E = 160000
D_IN = 256
D_H = 512
D_OUT = 256

NC = 2
NS = 16
L = 16

NPAD = 10240
COLS_PER_TILE = NPAD // NS

BATCH = 128
NB = -(-(E + N) // (NS * BATCH))
E2 = NS * NB * BATCH
ACC_ROWS = NPAD
ROWS_PER_TILE = N // NS
RED_BATCH = 25
IDXCHUNK = 28
NCHUNK = NB // IDXCHUNK
EPAD = E2 - (E + N)


def _rsqrt_newton(d):
  xi = plsc.bitcast(d, jnp.int32)
  i = jnp.int32(0x5F3759DF) - lax.shift_right_logical(xi, 1)
  r = plsc.bitcast(i, jnp.float32)
  for _ in range(3):
    r = r * (1.5 - 0.5 * d * r * r)
  return r


EPT_A = E // NS


def _stats_body(src_hbm, dst_hbm, dis_hbm, w_hbm,
                part_v, idx_v, idx2_v, disfull_v, red_v, chunk_v,
                stage_sh, dis_sh):
  c = lax.axis_index("c")
  s = lax.axis_index("s")

  @pl.when(c == 0)
  def _():
    zeros16 = jnp.zeros((L,), jnp.float32)
    ones16 = jnp.ones((L,), jnp.float32)

    def zero_part(i, _):
      part_v[pl.ds(i * L, L)] = zeros16
      return 0
    lax.fori_loop(0, NPAD // L, zero_part, 0)

    pltpu.sync_copy(dst_hbm.at[pl.ds(s * EPT_A, EPT_A)], idx_v)

    def deg_step(i, _):
      d_idx = idx_v[pl.ds(i * L, L)]
      plsc.addupdate_scatter(part_v, [d_idx], ones16)
      return 0
    lax.fori_loop(0, EPT_A // L, deg_step, 0)

    pltpu.sync_copy(part_v, stage_sh.at[s])
    plsc.subcore_barrier()

    col0 = s * COLS_PER_TILE
    pltpu.sync_copy(stage_sh.at[:, pl.ds(col0, COLS_PER_TILE)], red_v)

    def dis_step(k, _):
      acc = red_v[0, pl.ds(k * L, L)]
      for t in range(1, NS):
        acc = acc + red_v[t, pl.ds(k * L, L)]
      chunk_v[pl.ds(k * L, L)] = _rsqrt_newton(acc + 1.0)
      return 0
    lax.fori_loop(0, COLS_PER_TILE // L, dis_step, 0)

    pltpu.sync_copy(chunk_v, dis_hbm.at[pl.ds(col0, COLS_PER_TILE)])
    pltpu.sync_copy(chunk_v, dis_sh.at[pl.ds(col0, COLS_PER_TILE)])
    plsc.subcore_barrier()

    pltpu.sync_copy(dis_sh, disfull_v)
    pltpu.sync_copy(src_hbm.at[pl.ds(s * EPT_A, EPT_A)], idx2_v)

    lax.fori_loop(0, NPAD // L, zero_part, 0)

    def t_step(i, _):
      d_idx = idx_v[pl.ds(i * L, L)]
      s_idx = idx2_v[pl.ds(i * L, L)]
      dvals = plsc.load_gather(disfull_v, [d_idx])
      plsc.addupdate_scatter(part_v, [s_idx], dvals)
      return 0
    lax.fori_loop(0, EPT_A // L, t_step, 0)

    pltpu.sync_copy(part_v, stage_sh.at[s])
    plsc.subcore_barrier()

    pltpu.sync_copy(stage_sh.at[:, pl.ds(col0, COLS_PER_TILE)], red_v)

    def w_step(k, _):
      acc = red_v[0, pl.ds(k * L, L)]
      for t in range(1, NS):
        acc = acc + red_v[t, pl.ds(k * L, L)]
      dv = disfull_v[pl.ds(col0 + k * L, L)]
      chunk_v[pl.ds(k * L, L)] = dv * (acc + dv)
      return 0
    lax.fori_loop(0, COLS_PER_TILE // L, w_step, 0)

    pltpu.sync_copy(chunk_v, w_hbm.at[pl.ds(col0, COLS_PER_TILE)])


@functools.partial(jax.jit, static_argnames=())
def _sc_stats(src, dst):
  mesh = plsc.VectorSubcoreMesh(core_axis_name="c", subcore_axis_name="s")
  f = pl.kernel(
      _stats_body,
      out_type=(
          jax.ShapeDtypeStruct((NPAD,), jnp.float32),
          jax.ShapeDtypeStruct((NPAD,), jnp.float32),
      ),
      mesh=mesh,
      scratch_types=[
          pltpu.VMEM((NPAD,), jnp.float32),
          pltpu.VMEM((EPT_A,), jnp.int32),
          pltpu.VMEM((EPT_A,), jnp.int32),
          pltpu.VMEM((NPAD,), jnp.float32),
          pltpu.VMEM((NS, COLS_PER_TILE), jnp.float32),
          pltpu.VMEM((COLS_PER_TILE,), jnp.float32),
          pltpu.VMEM_SHARED((NS, NPAD), jnp.float32),
          pltpu.VMEM_SHARED((NPAD,), jnp.float32),
      ],
      compiler_params=pltpu.CompilerParams(use_tc_tiling_on_sc=False, needs_layout_passes=False),
  )
  return f(src, dst)


BM = 400


def _mm_body(x_ref, w_ref, dis_ref, out_ref):
  y = dis_ref[...] * jnp.dot(
      x_ref[...].astype(jnp.bfloat16), w_ref[...].astype(jnp.bfloat16),
      preferred_element_type=jnp.float32)
  out_ref[...] = y.astype(jnp.bfloat16).reshape(2 * BM, D_H // 2)


def _tc_scaled_mm(x, W1, dis2):
  return pl.pallas_call(
      _mm_body,
      out_shape=jax.ShapeDtypeStruct((NC * N, D_H // 2), jnp.bfloat16),
      grid=(N // BM,),
      in_specs=[
          pl.BlockSpec((BM, D_IN), lambda i: (i, 0)),
          pl.BlockSpec((D_IN, D_H), lambda i: (0, 0)),
          pl.BlockSpec((BM, 1), lambda i: (i, 0)),
      ],
      out_specs=pl.BlockSpec((2 * BM, D_H // 2), lambda i: (i, 0)),
  )(x, W1, dis2)


SLAB = D_H // NC


def _msg_body(src_hbm, dst_hbm, y2_hbm, out_hbm,
              gidx_v, didx_v, buf0, buf1, zbuf,
              sem0, sem1, sem2, sem3, acc_sh):
  c = lax.axis_index("c")
  s = lax.axis_index("s")
  iota16 = lax.iota(jnp.int32, L)
  CH = IDXCHUNK * BATCH
  zeros32 = jnp.zeros((2 * L,), jnp.bfloat16)

  def zb(i, _):
    for f in range(SLAB // (2 * L)):
      zbuf[i, pl.ds(f * 2 * L, 2 * L)] = zeros32
    return 0
  lax.fori_loop(0, 16, zb, 0)

  for z in range(COLS_PER_TILE // 16):
    pltpu.sync_copy(zbuf, acc_sh.at[pl.ds(s * COLS_PER_TILE + z * 16, 16)])
  plsc.subcore_barrier()

  def fire_g(j, buf, sem):
    pltpu.async_copy(y2_hbm.at[gidx_v.at[pl.ds(j * BATCH, BATCH)]], buf, sem)

  def wait_g_fire_s(j, buf, gsem, ssem):
    pltpu.make_async_copy(
        y2_hbm.at[gidx_v.at[pl.ds(j * BATCH, BATCH)]], buf, gsem).wait()
    pltpu.async_copy(
        buf, acc_sh.at[didx_v.at[pl.ds(j * BATCH, BATCH)]], ssem, add=True)

  def wait_s(j, buf, ssem):
    pltpu.make_async_copy(
        buf, acc_sh.at[didx_v.at[pl.ds(j * BATCH, BATCH)]], ssem).wait()

  def chunk_step(ic, _):
    w0 = (s * NB + ic * IDXCHUNK) * BATCH

    @pl.when(w0 + CH <= E)
    def _():
      pltpu.sync_copy(src_hbm.at[pl.ds(w0, CH)], gidx_v)
      pltpu.sync_copy(dst_hbm.at[pl.ds(w0, CH)], didx_v)

    @pl.when((w0 < E) & (w0 + CH > E))
    def _():
      rem = E % CH
      pltpu.sync_copy(src_hbm.at[pl.ds(E - rem, rem)],
                      gidx_v.at[pl.ds(0, rem)])
      pltpu.sync_copy(dst_hbm.at[pl.ds(E - rem, rem)],
                      didx_v.at[pl.ds(0, rem)])

    def asm_step(i, _):
      g = w0 + i * L + iota16
      sv = gidx_v[pl.ds(i * L, L)]
      dv = didx_v[pl.ds(i * L, L)]
      selfid = g - E
      src_val = jnp.where(g < E, sv, jnp.where(g < E + N, selfid, 0))
      dst_val = jnp.where(g < E, dv, jnp.where(g < E + N, selfid, N))
      gidx_v[pl.ds(i * L, L)] = src_val * NC + c
      didx_v[pl.ds(i * L, L)] = dst_val
      return 0
    lax.fori_loop(0, CH // L, asm_step, 0)
    fire_g(0, buf0, sem0)
    fire_g(1, buf1, sem1)

    def edge_step(t, _):
      j = t * 2
      wait_g_fire_s(j, buf0, sem0, sem2)
      wait_g_fire_s(j + 1, buf1, sem1, sem3)
      wait_s(j, buf0, sem2)

      @pl.when(j + 2 < IDXCHUNK)
      def _():
        fire_g(j + 2, buf0, sem0)

      wait_s(j + 1, buf1, sem3)

      @pl.when(j + 3 < IDXCHUNK)
      def _():
        fire_g(j + 3, buf1, sem1)

      return 0
    lax.fori_loop(0, IDXCHUNK // 2, edge_step, 0)
    return 0
  lax.fori_loop(0, NCHUNK, chunk_step, 0)

  plsc.subcore_barrier()

  pltpu.sync_copy(acc_sh.at[pl.ds(s * COLS_PER_TILE, COLS_PER_TILE)],
                  out_hbm.at[c, pl.ds(s * COLS_PER_TILE, COLS_PER_TILE)])


def _sc_message(src, dst, y2):
  mesh = plsc.VectorSubcoreMesh(core_axis_name="c", subcore_axis_name="s")
  f = pl.kernel(
      _msg_body,
      out_type=jax.ShapeDtypeStruct((NC, NPAD, SLAB), jnp.bfloat16),
      mesh=mesh,
      scratch_types=[
          pltpu.VMEM((IDXCHUNK * BATCH,), jnp.int32),
          pltpu.VMEM((IDXCHUNK * BATCH,), jnp.int32),
          pltpu.VMEM((BATCH, SLAB), jnp.bfloat16),
          pltpu.VMEM((BATCH, SLAB), jnp.bfloat16),
          pltpu.VMEM((16, SLAB), jnp.bfloat16),
          pltpu.SemaphoreType.DMA,
          pltpu.SemaphoreType.DMA,
          pltpu.SemaphoreType.DMA,
          pltpu.SemaphoreType.DMA,
          pltpu.VMEM_SHARED((ACC_ROWS, SLAB), jnp.bfloat16),
      ],
      compiler_params=pltpu.CompilerParams(use_tc_tiling_on_sc=False, needs_layout_passes=False),
  )
  return f(src, dst, y2)


BMD = 400


HALF = SLAB // 2


def _red_body(acc_ref, dis_ref, w_ref, b1_ref, w2_ref, b2_ref, out_ref, s_scr):
  i = pl.program_id(0)

  @pl.when(i == 0)
  def _():
    s_scr[...] = jnp.zeros_like(s_scr)

  dis = dis_ref[...]
  wv = w_ref[...]
  for p in range(NC):
    h = jnp.maximum(
        dis * acc_ref[p].astype(jnp.float32) + b1_ref[p], 0.0)
    s_scr[p] += jnp.sum(wv * h, axis=0, keepdims=True)

  @pl.when(i == (N // BMD) - 1)
  def _():
    acc = jnp.zeros((1, D_OUT), jnp.float32)
    for p in range(NC):
      acc = acc + jnp.dot(s_scr[p], w2_ref[p],
                          preferred_element_type=jnp.float32)
    out_ref[...] = acc * (1.0 / N) + b2_ref[...]


def _tc_reduce(acc2, dis2, w2col, b1r, W2r, b2r):
  return pl.pallas_call(
      _red_body,
      out_shape=jax.ShapeDtypeStruct((1, D_OUT), jnp.float32),
      grid=(N // BMD,),
      in_specs=[
          pl.BlockSpec((NC, BMD, SLAB), lambda i: (0, i, 0)),
          pl.BlockSpec((BMD, 1), lambda i: (i, 0)),
          pl.BlockSpec((BMD, 1), lambda i: (i, 0)),
          pl.BlockSpec((NC, 1, SLAB), lambda i: (0, 0, 0)),
          pl.BlockSpec((NC, SLAB, D_OUT), lambda i: (0, 0, 0)),
          pl.BlockSpec((1, D_OUT), lambda i: (0, 0)),
      ],
      out_specs=pl.BlockSpec((1, D_OUT), lambda i: (0, 0)),
      scratch_shapes=[pltpu.VMEM((NC, 1, SLAB), jnp.float32)],
  )(acc2, dis2, w2col, b1r, W2r, b2r)


def kernel(x, edge_index, W1, b1, W2, b2):
  src = edge_index[0]
  dst = edge_index[1]

  dis_pad, w_pad = _sc_stats(src, dst)
  dis = dis_pad[:N]

  y2 = _tc_scaled_mm(x, W1, dis.reshape(N, 1))

  acc2 = _sc_message(src, dst, y2)

  out = _tc_reduce(acc2,
                   dis.reshape(N, 1),
                   w_pad[:N].reshape(N, 1),
                   b1.reshape(NC, 1, SLAB),
                   W2.reshape(NC, SLAB, D_OUT),
                   b2.reshape(1, D_OUT))
  return out.reshape(D_OUT)


# speedup vs baseline: 1.1656x; 1.0032x over previous
import functools

import jax
import jax.numpy as jnp
from jax import lax
from jax.experimental import pallas as pl
from jax.experimental.pallas import tpu as pltpu
from jax.experimental.pallas import tpu_sc as plsc

N = 10000
E = 160000
D_IN = 256
D_H = 512
D_OUT = 256

NC = 2
NS = 16
L = 16

NPAD = 10240
COLS_PER_TILE = NPAD // NS

BATCH = 128
NB = -(-(E + N) // (NS * BATCH))
E2 = NS * NB * BATCH
ACC_ROWS = NPAD
ROWS_PER_TILE = N // NS
RED_BATCH = 25
IDXCHUNK = 42
NCHUNK = NB // IDXCHUNK
EPAD = E2 - (E + N)


def _rsqrt_newton(d):
  xi = plsc.bitcast(d, jnp.int32)
  i = jnp.int32(0x5F3759DF) - lax.shift_right_logical(xi, 1)
  r = plsc.bitcast(i, jnp.float32)
  for _ in range(3):
    r = r * (1.5 - 0.5 * d * r * r)
  return r


EPT_A = E // NS


def _stats_body(src_hbm, dst_hbm, dis_hbm, w_hbm,
                part_v, idx_v, idx2_v, disfull_v, red_v, chunk_v,
                stage_sh, dis_sh):
  c = lax.axis_index("c")
  s = lax.axis_index("s")

  @pl.when(c == 0)
  def _():
    zeros16 = jnp.zeros((L,), jnp.float32)
    ones16 = jnp.ones((L,), jnp.float32)

    def zero_part(i, _):
      part_v[pl.ds(i * L, L)] = zeros16
      return 0
    lax.fori_loop(0, NPAD // L, zero_part, 0)

    pltpu.sync_copy(dst_hbm.at[pl.ds(s * EPT_A, EPT_A)], idx_v)

    def deg_step(i, _):
      d_idx = idx_v[pl.ds(i * L, L)]
      plsc.addupdate_scatter(part_v, [d_idx], ones16)
      return 0
    lax.fori_loop(0, EPT_A // L, deg_step, 0)

    pltpu.sync_copy(part_v, stage_sh.at[s])
    plsc.subcore_barrier()

    col0 = s * COLS_PER_TILE
    pltpu.sync_copy(stage_sh.at[:, pl.ds(col0, COLS_PER_TILE)], red_v)

    def dis_step(k, _):
      acc = red_v[0, pl.ds(k * L, L)]
      for t in range(1, NS):
        acc = acc + red_v[t, pl.ds(k * L, L)]
      chunk_v[pl.ds(k * L, L)] = _rsqrt_newton(acc + 1.0)
      return 0
    lax.fori_loop(0, COLS_PER_TILE // L, dis_step, 0)

    pltpu.sync_copy(chunk_v, dis_hbm.at[pl.ds(col0, COLS_PER_TILE)])
    pltpu.sync_copy(chunk_v, dis_sh.at[pl.ds(col0, COLS_PER_TILE)])
    plsc.subcore_barrier()

    pltpu.sync_copy(dis_sh, disfull_v)
    pltpu.sync_copy(src_hbm.at[pl.ds(s * EPT_A, EPT_A)], idx2_v)

    lax.fori_loop(0, NPAD // L, zero_part, 0)

    def t_step(i, _):
      d_idx = idx_v[pl.ds(i * L, L)]
      s_idx = idx2_v[pl.ds(i * L, L)]
      dvals = plsc.load_gather(disfull_v, [d_idx])
      plsc.addupdate_scatter(part_v, [s_idx], dvals)
      return 0
    lax.fori_loop(0, EPT_A // L, t_step, 0)

    pltpu.sync_copy(part_v, stage_sh.at[s])
    plsc.subcore_barrier()

    pltpu.sync_copy(stage_sh.at[:, pl.ds(col0, COLS_PER_TILE)], red_v)

    def w_step(k, _):
      acc = red_v[0, pl.ds(k * L, L)]
      for t in range(1, NS):
        acc = acc + red_v[t, pl.ds(k * L, L)]
      dv = disfull_v[pl.ds(col0 + k * L, L)]
      chunk_v[pl.ds(k * L, L)] = dv * (acc + dv)
      return 0
    lax.fori_loop(0, COLS_PER_TILE // L, w_step, 0)

    pltpu.sync_copy(chunk_v, w_hbm.at[pl.ds(col0, COLS_PER_TILE)])


@functools.partial(jax.jit, static_argnames=())
def _sc_stats(src, dst):
  mesh = plsc.VectorSubcoreMesh(core_axis_name="c", subcore_axis_name="s")
  f = pl.kernel(
      _stats_body,
      out_type=(
          jax.ShapeDtypeStruct((NPAD,), jnp.float32),
          jax.ShapeDtypeStruct((NPAD,), jnp.float32),
      ),
      mesh=mesh,
      scratch_types=[
          pltpu.VMEM((NPAD,), jnp.float32),
          pltpu.VMEM((EPT_A,), jnp.int32),
          pltpu.VMEM((EPT_A,), jnp.int32),
          pltpu.VMEM((NPAD,), jnp.float32),
          pltpu.VMEM((NS, COLS_PER_TILE), jnp.float32),
          pltpu.VMEM((COLS_PER_TILE,), jnp.float32),
          pltpu.VMEM_SHARED((NS, NPAD), jnp.float32),
          pltpu.VMEM_SHARED((NPAD,), jnp.float32),
      ],
      compiler_params=pltpu.CompilerParams(use_tc_tiling_on_sc=False, needs_layout_passes=False),
  )
  return f(src, dst)


BM = 400


def _mm_body(x_ref, w_ref, dis_ref, out_ref):
  y = dis_ref[...] * jnp.dot(
      x_ref[...].astype(jnp.bfloat16), w_ref[...].astype(jnp.bfloat16),
      preferred_element_type=jnp.float32)
  out_ref[...] = y.astype(jnp.bfloat16).reshape(2 * BM, D_H // 2)


def _tc_scaled_mm(x, W1, dis2):
  return pl.pallas_call(
      _mm_body,
      out_shape=jax.ShapeDtypeStruct((NC * N, D_H // 2), jnp.bfloat16),
      grid=(N // BM,),
      in_specs=[
          pl.BlockSpec((BM, D_IN), lambda i: (i, 0)),
          pl.BlockSpec((D_IN, D_H), lambda i: (0, 0)),
          pl.BlockSpec((BM, 1), lambda i: (i, 0)),
      ],
      out_specs=pl.BlockSpec((2 * BM, D_H // 2), lambda i: (i, 0)),
  )(x, W1, dis2)


SLAB = D_H // NC


def _msg_body(src_hbm, dst_hbm, y2_hbm, out_hbm,
              gidx_v, didx_v, buf0, buf1, zbuf,
              sem0, sem1, sem2, sem3, acc_sh):
  c = lax.axis_index("c")
  s = lax.axis_index("s")
  iota16 = lax.iota(jnp.int32, L)
  CH = IDXCHUNK * BATCH
  zeros32 = jnp.zeros((2 * L,), jnp.bfloat16)

  def zb(i, _):
    for f in range(SLAB // (2 * L)):
      zbuf[i, pl.ds(f * 2 * L, 2 * L)] = zeros32
    return 0
  lax.fori_loop(0, 16, zb, 0)

  for z in range(COLS_PER_TILE // 16):
    pltpu.sync_copy(zbuf, acc_sh.at[pl.ds(s * COLS_PER_TILE + z * 16, 16)])
  plsc.subcore_barrier()

  def fire_g(j, buf, sem):
    pltpu.async_copy(y2_hbm.at[gidx_v.at[pl.ds(j * BATCH, BATCH)]], buf, sem)

  def wait_g_fire_s(j, buf, gsem, ssem):
    pltpu.make_async_copy(
        y2_hbm.at[gidx_v.at[pl.ds(j * BATCH, BATCH)]], buf, gsem).wait()
    pltpu.async_copy(
        buf, acc_sh.at[didx_v.at[pl.ds(j * BATCH, BATCH)]], ssem, add=True)

  def wait_s(j, buf, ssem):
    pltpu.make_async_copy(
        buf, acc_sh.at[didx_v.at[pl.ds(j * BATCH, BATCH)]], ssem).wait()

  def chunk_step(ic, _):
    w0 = (s * NB + ic * IDXCHUNK) * BATCH

    @pl.when(w0 + CH <= E)
    def _():
      pltpu.sync_copy(src_hbm.at[pl.ds(w0, CH)], gidx_v)
      pltpu.sync_copy(dst_hbm.at[pl.ds(w0, CH)], didx_v)

    @pl.when((w0 < E) & (w0 + CH > E))
    def _():
      rem = E % CH
      pltpu.sync_copy(src_hbm.at[pl.ds(E - rem, rem)],
                      gidx_v.at[pl.ds(0, rem)])
      pltpu.sync_copy(dst_hbm.at[pl.ds(E - rem, rem)],
                      didx_v.at[pl.ds(0, rem)])

    def asm_step(i, _):
      g = w0 + i * L + iota16
      sv = gidx_v[pl.ds(i * L, L)]
      dv = didx_v[pl.ds(i * L, L)]
      selfid = g - E
      src_val = jnp.where(g < E, sv, jnp.where(g < E + N, selfid, 0))
      dst_val = jnp.where(g < E, dv, jnp.where(g < E + N, selfid, N))
      gidx_v[pl.ds(i * L, L)] = src_val * NC + c
      didx_v[pl.ds(i * L, L)] = dst_val
      return 0
    lax.fori_loop(0, CH // L, asm_step, 0)
    fire_g(0, buf0, sem0)
    fire_g(1, buf1, sem1)

    def edge_step(t, _):
      j = t * 2
      wait_g_fire_s(j, buf0, sem0, sem2)
      wait_g_fire_s(j + 1, buf1, sem1, sem3)
      wait_s(j, buf0, sem2)

      @pl.when(j + 2 < IDXCHUNK)
      def _():
        fire_g(j + 2, buf0, sem0)

      wait_s(j + 1, buf1, sem3)

      @pl.when(j + 3 < IDXCHUNK)
      def _():
        fire_g(j + 3, buf1, sem1)

      return 0
    lax.fori_loop(0, IDXCHUNK // 2, edge_step, 0)
    return 0
  lax.fori_loop(0, NCHUNK, chunk_step, 0)

  plsc.subcore_barrier()

  pltpu.sync_copy(acc_sh.at[pl.ds(s * COLS_PER_TILE, COLS_PER_TILE)],
                  out_hbm.at[c, pl.ds(s * COLS_PER_TILE, COLS_PER_TILE)])


def _sc_message(src, dst, y2):
  mesh = plsc.VectorSubcoreMesh(core_axis_name="c", subcore_axis_name="s")
  f = pl.kernel(
      _msg_body,
      out_type=jax.ShapeDtypeStruct((NC, NPAD, SLAB), jnp.bfloat16),
      mesh=mesh,
      scratch_types=[
          pltpu.VMEM((IDXCHUNK * BATCH,), jnp.int32),
          pltpu.VMEM((IDXCHUNK * BATCH,), jnp.int32),
          pltpu.VMEM((BATCH, SLAB), jnp.bfloat16),
          pltpu.VMEM((BATCH, SLAB), jnp.bfloat16),
          pltpu.VMEM((16, SLAB), jnp.bfloat16),
          pltpu.SemaphoreType.DMA,
          pltpu.SemaphoreType.DMA,
          pltpu.SemaphoreType.DMA,
          pltpu.SemaphoreType.DMA,
          pltpu.VMEM_SHARED((ACC_ROWS, SLAB), jnp.bfloat16),
      ],
      compiler_params=pltpu.CompilerParams(use_tc_tiling_on_sc=False, needs_layout_passes=False),
  )
  return f(src, dst, y2)


BMD = 400


HALF = SLAB // 2


def _red_body(acc_ref, dis_ref, w_ref, b1_ref, w2_ref, b2_ref, out_ref, s_scr):
  i = pl.program_id(0)

  @pl.when(i == 0)
  def _():
    s_scr[...] = jnp.zeros_like(s_scr)

  dis = dis_ref[...]
  wv = w_ref[...]
  for p in range(NC):
    h = jnp.maximum(
        dis * acc_ref[p].astype(jnp.float32) + b1_ref[p], 0.0)
    s_scr[p] += jnp.sum(wv * h, axis=0, keepdims=True)

  @pl.when(i == (N // BMD) - 1)
  def _():
    acc = jnp.zeros((1, D_OUT), jnp.float32)
    for p in range(NC):
      acc = acc + jnp.dot(s_scr[p], w2_ref[p],
                          preferred_element_type=jnp.float32)
    out_ref[...] = acc * (1.0 / N) + b2_ref[...]


def _tc_reduce(acc2, dis2, w2col, b1r, W2r, b2r):
  return pl.pallas_call(
      _red_body,
      out_shape=jax.ShapeDtypeStruct((1, D_OUT), jnp.float32),
      grid=(N // BMD,),
      in_specs=[
          pl.BlockSpec((NC, BMD, SLAB), lambda i: (0, i, 0)),
          pl.BlockSpec((BMD, 1), lambda i: (i, 0)),
          pl.BlockSpec((BMD, 1), lambda i: (i, 0)),
          pl.BlockSpec((NC, 1, SLAB), lambda i: (0, 0, 0)),
          pl.BlockSpec((NC, SLAB, D_OUT), lambda i: (0, 0, 0)),
          pl.BlockSpec((1, D_OUT), lambda i: (0, 0)),
      ],
      out_specs=pl.BlockSpec((1, D_OUT), lambda i: (0, 0)),
      scratch_shapes=[pltpu.VMEM((NC, 1, SLAB), jnp.float32)],
  )(acc2, dis2, w2col, b1r, W2r, b2r)


def kernel(x, edge_index, W1, b1, W2, b2):
  src = edge_index[0]
  dst = edge_index[1]

  dis_pad, w_pad = _sc_stats(src, dst)
  dis = dis_pad[:N]

  y2 = _tc_scaled_mm(x, W1, dis.reshape(N, 1))

  acc2 = _sc_message(src, dst, y2)

  out = _tc_reduce(acc2,
                   dis.reshape(N, 1),
                   w_pad[:N].reshape(N, 1),
                   b1.reshape(NC, 1, SLAB),
                   W2.reshape(NC, SLAB, D_OUT),
                   b2.reshape(1, D_OUT))
  return out.reshape(D_OUT)
